# Initial kernel scaffold; baseline (speedup 1.0000x reference)
import functools

import jax
import jax.numpy as jnp
from jax import lax
from jax.experimental import pallas as pl
from jax.experimental.pallas import tpu as pltpu
from jax.experimental.pallas import tpu_sc as plsc

N_NODES = 10000
N_EDGES = 160000
N_HIS = 6
HID = 128
EMB = 16
NTYPES = 9
LAYERS = 5
RADIUS = 0.1
EPS = 1e-5

NC = 2
NS = 16
NW = NC * NS

CH = 128
E_PAD = 163840
PER_W = E_PAD // NW
NCHUNK = PER_W // CH

N_PAD = 10240
ROWS_PER_TILE = N_PAD // NS

_F32 = jnp.float32


def _sc_mesh():
    return plsc.VectorSubcoreMesh(
        core_axis_name="c", subcore_axis_name="s", num_cores=NC, num_subcores=NS
    )


def _gather_pair(tab_a, tab_b, idx_a, idx_b):
    d = tab_a.shape[-1]

    def body(ta, tb, ia, ib, oa, ob, iva, ivb, rva, rvb, sema, semb):
        c = lax.axis_index("c")
        s = lax.axis_index("s")
        base = (s * NC + c) * PER_W

        def step(i, carry):
            off = base + i * CH
            pltpu.sync_copy(ia.at[pl.ds(off, CH)], iva)
            pltpu.sync_copy(ib.at[pl.ds(off, CH)], ivb)
            cpa = pltpu.async_copy(ta.at[iva], rva, sema)
            cpb = pltpu.async_copy(tb.at[ivb], rvb, semb)
            cpa.wait()
            cpb.wait()
            pltpu.sync_copy(rva, oa.at[pl.ds(off, CH)])
            pltpu.sync_copy(rvb, ob.at[pl.ds(off, CH)])
            return carry

        lax.fori_loop(0, NCHUNK, step, 0)

    f = pl.kernel(
        body,
        out_type=(
            jax.ShapeDtypeStruct((E_PAD, d), _F32),
            jax.ShapeDtypeStruct((E_PAD, d), _F32),
        ),
        mesh=_sc_mesh(),
        scratch_types=[
            pltpu.VMEM((CH,), jnp.int32),
            pltpu.VMEM((CH,), jnp.int32),
            pltpu.VMEM((CH, d), _F32),
            pltpu.VMEM((CH, d), _F32),
            pltpu.SemaphoreType.DMA,
            pltpu.SemaphoreType.DMA,
        ],
    )
    return f(tab_a, tab_b, idx_a, idx_b)


def _segment_sum(vals, idx, zeros_init):
    def body(vh, ih, zh, oh, iv, rv, acc):
        c = lax.axis_index("c")
        s = lax.axis_index("s")
        r0 = s * ROWS_PER_TILE
        pltpu.sync_copy(zh.at[pl.ds(r0, ROWS_PER_TILE)], acc.at[pl.ds(r0, ROWS_PER_TILE)])
        plsc.subcore_barrier()

        base = (s * NC + c) * PER_W

        def step(i, carry):
            off = base + i * CH
            pltpu.sync_copy(ih.at[pl.ds(off, CH)], iv)
            pltpu.sync_copy(vh.at[pl.ds(off, CH)], rv)
            pltpu.sync_copy(rv, acc.at[iv], add=True)
            return carry

        lax.fori_loop(0, NCHUNK, step, 0)
        plsc.subcore_barrier()
        pltpu.sync_copy(
            acc.at[pl.ds(r0, ROWS_PER_TILE)],
            oh.at[pl.ds(c * N_PAD + r0, ROWS_PER_TILE)],
        )

    f = pl.kernel(
        body,
        out_type=jax.ShapeDtypeStruct((NC * N_PAD, HID), _F32),
        mesh=_sc_mesh(),
        scratch_types=[
            pltpu.VMEM((CH,), jnp.int32),
            pltpu.VMEM((CH, HID), _F32),
            pltpu.VMEM_SHARED((N_PAD, HID), _F32),
        ],
    )
    return f(vals, idx, zeros_init)


def _dot(a, b):
    return jnp.dot(a, b, preferred_element_type=_F32)


def _mlp_tail(h, w1, w2, b1, b2, g, be):
    h = jnp.maximum(h, 0.0)
    h = jnp.maximum(_dot(h, w1) + b1, 0.0)
    h = _dot(h, w2) + b2
    mu = jnp.mean(h, axis=-1, keepdims=True)
    var = jnp.mean((h - mu) * (h - mu), axis=-1, keepdims=True)
    return (h - mu) * lax.rsqrt(var + EPS) * g + be


def _pack_bias(*rows):
    mat = jnp.stack(list(rows) + [jnp.zeros((HID,), _F32)] * (8 - len(rows)))
    return mat


_BE = 2048
_BN = 1000


def _full(shape):
    return pl.BlockSpec(shape, lambda i: tuple(0 for _ in shape))


def _rows(shape):
    return pl.BlockSpec(shape, lambda i: (i,) + tuple(0 for _ in shape[1:]))


def _edge_enc(ps, pr, w0p, w1, w2, bias):
    def body(ps_ref, pr_ref, w0_ref, w1_ref, w2_ref, bb_ref, o_ref):
        dx = (ps_ref[:, 0:1] - pr_ref[:, 0:1]) / RADIUS
        dy = (ps_ref[:, 1:2] - pr_ref[:, 1:2]) / RADIUS
        dist = jnp.sqrt(dx * dx + dy * dy)
        w0 = w0_ref[...]
        bb = bb_ref[...]
        h = dx * w0[0:1] + dy * w0[1:2] + dist * w0[2:3] + bb[0:1]
        o_ref[...] = _mlp_tail(h, w1_ref[...], w2_ref[...], bb[1:2], bb[2:3],
                               bb[3:4], bb[4:5])

    return pl.pallas_call(
        body,
        grid=(E_PAD // _BE,),
        in_specs=[
            _rows((_BE, 16)),
            _rows((_BE, 16)),
            _full((8, HID)),
            _full((HID, HID)),
            _full((HID, HID)),
            _full((8, HID)),
        ],
        out_specs=_rows((_BE, HID)),
        out_shape=jax.ShapeDtypeStruct((E_PAD, HID), _F32),
    )(ps, pr, w0p, w1, w2, bias)


def _node_enc(x, w0p, w1, w2, bias, ws_next, wr_next):
    def body(x_ref, w0_ref, w1_ref, w2_ref, bb_ref, ws_ref, wr_ref,
             on_ref, os_ref, or_ref):
        bb = bb_ref[...]
        h = _dot(x_ref[...], w0_ref[...]) + bb[0:1]
        y = _mlp_tail(h, w1_ref[...], w2_ref[...], bb[1:2], bb[2:3],
                      bb[3:4], bb[4:5])
        on_ref[...] = y
        os_ref[...] = _dot(y, ws_ref[...])
        or_ref[...] = _dot(y, wr_ref[...])

    sds = jax.ShapeDtypeStruct((N_NODES, HID), _F32)
    return pl.pallas_call(
        body,
        grid=(N_NODES // _BN,),
        in_specs=[
            _rows((_BN, 24)),
            _full((24, HID)),
            _full((HID, HID)),
            _full((HID, HID)),
            _full((8, HID)),
            _full((HID, HID)),
            _full((HID, HID)),
        ],
        out_specs=[_rows((_BN, HID))] * 3,
        out_shape=[sds, sds, sds],
    )(x, w0p, w1, w2, bias, ws_next, wr_next)


def _edge_update(edges, gs, gr, we, w1, w2, bias):
    def body(e_ref, gs_ref, gr_ref, we_ref, w1_ref, w2_ref, bb_ref, o_ref):
        bb = bb_ref[...]
        x = e_ref[...]
        h = _dot(x, we_ref[...]) + gs_ref[...] + gr_ref[...] + bb[0:1]
        y = _mlp_tail(h, w1_ref[...], w2_ref[...], bb[1:2], bb[2:3],
                      bb[3:4], bb[4:5])
        out = x + y
        row = lax.broadcasted_iota(jnp.int32, (_BE, 1), 0) + pl.program_id(0) * _BE
        o_ref[...] = jnp.where(row < N_EDGES, out, 0.0)

    return pl.pallas_call(
        body,
        grid=(E_PAD // _BE,),
        in_specs=[
            _rows((_BE, HID)),
            _rows((_BE, HID)),
            _rows((_BE, HID)),
            _full((HID, HID)),
            _full((HID, HID)),
            _full((HID, HID)),
            _full((8, HID)),
        ],
        out_specs=_rows((_BE, HID)),
        out_shape=jax.ShapeDtypeStruct((E_PAD, HID), _F32),
    )(edges, gs, gr, we, w1, w2, bias)


def _node_update(nodes, agg, wn, wa, w1, w2, bias, ws_next, wr_next):
    def body(n_ref, a_ref, wn_ref, wa_ref, w1_ref, w2_ref, bb_ref,
             ws_ref, wr_ref, on_ref, os_ref, or_ref):
        bb = bb_ref[...]
        x = n_ref[...]
        a = a_ref[0] + a_ref[1]
        h = _dot(x, wn_ref[...]) + _dot(a, wa_ref[...]) + bb[0:1]
        y = _mlp_tail(h, w1_ref[...], w2_ref[...], bb[1:2], bb[2:3],
                      bb[3:4], bb[4:5])
        nn = x + y
        on_ref[...] = nn
        os_ref[...] = _dot(nn, ws_ref[...])
        or_ref[...] = _dot(nn, wr_ref[...])

    sds = jax.ShapeDtypeStruct((N_NODES, HID), _F32)
    return pl.pallas_call(
        body,
        grid=(N_NODES // _BN,),
        in_specs=[
            _rows((_BN, HID)),
            pl.BlockSpec((NC, _BN, HID), lambda i: (0, i, 0)),
            _full((HID, HID)),
            _full((HID, HID)),
            _full((HID, HID)),
            _full((HID, HID)),
            _full((8, HID)),
            _full((HID, HID)),
            _full((HID, HID)),
        ],
        out_specs=[_rows((_BN, HID))] * 3,
        out_shape=[sds, sds, sds],
    )(nodes, agg, wn, wa, w1, w2, bias, ws_next, wr_next)


def _node_update_dec(nodes, agg, wn, wa, w1, w2, bias, d0, d1, d2p, dbias):
    def body(n_ref, a_ref, wn_ref, wa_ref, w1_ref, w2_ref, bb_ref,
             d0_ref, d1_ref, d2_ref, db_ref, o_ref):
        bb = bb_ref[...]
        db = db_ref[...]
        x = n_ref[...]
        a = a_ref[0] + a_ref[1]
        h = _dot(x, wn_ref[...]) + _dot(a, wa_ref[...]) + bb[0:1]
        y = _mlp_tail(h, w1_ref[...], w2_ref[...], bb[1:2], bb[2:3],
                      bb[3:4], bb[4:5])
        nn = x + y
        h = jnp.maximum(_dot(nn, d0_ref[...]) + db[0:1], 0.0)
        h = jnp.maximum(_dot(h, d1_ref[...]) + db[1:2], 0.0)
        o_ref[...] = _dot(h, d2_ref[...]) + db[2:3]

    return pl.pallas_call(
        body,
        grid=(N_NODES // _BN,),
        in_specs=[
            _rows((_BN, HID)),
            pl.BlockSpec((NC, _BN, HID), lambda i: (0, i, 0)),
            _full((HID, HID)),
            _full((HID, HID)),
            _full((HID, HID)),
            _full((HID, HID)),
            _full((8, HID)),
            _full((HID, HID)),
            _full((HID, HID)),
            _full((HID, HID)),
            _full((8, HID)),
        ],
        out_specs=_rows((_BN, HID)),
        out_shape=jax.ShapeDtypeStruct((N_NODES, HID), _F32),
    )(nodes, agg, wn, wa, w1, w2, bias, d0, d1, d2p, dbias)


def kernel(poss, particle_type, bounds, nonk_mask, tgt_poss, senders,
           receivers, params, num_rollouts):
    poss = poss + (jnp.asarray(num_rollouts) * 0).astype(poss.dtype)
    pos_last = poss[:, -1]

    vels = (poss[:, 1:] - poss[:, :-1]).reshape(N_NODES, (N_HIS - 1) * 2)
    d2w = jnp.concatenate([pos_last - bounds[:, 0], -pos_last + bounds[:, 1]], axis=1)
    d2w = jnp.clip(d2w / RADIUS, -1.0, 1.0)
    onehot = (particle_type[:, None] == jnp.arange(NTYPES)[None, :]).astype(_F32)
    x24 = jnp.concatenate([vels, d2w, onehot, jnp.zeros((N_NODES, 1), _F32)], axis=1)

    ne = params['node_enc']
    w0_eff = jnp.concatenate(
        [ne['W0'][:14], params['emb'] @ ne['W0'][14:30], jnp.zeros((1, HID), _F32)], axis=0)
    ne_bias = _pack_bias(ne['b0'], ne['b1'], ne['b2'], ne['g'], ne['be'])

    pad = jnp.zeros((E_PAD - N_EDGES,), jnp.int32)
    snd = jnp.concatenate([senders.astype(jnp.int32), pad])
    rcv = jnp.concatenate([receivers.astype(jnp.int32), pad])

    pos16 = jnp.pad(pos_last, ((0, 0), (0, 14)))
    ps, pr = _gather_pair(pos16, pos16, snd, rcv)

    ee = params['edge_enc']
    ee_w0 = jnp.concatenate([ee['W0'], jnp.zeros((5, HID), _F32)], axis=0)
    ee_bias = _pack_bias(ee['b0'], ee['b1'], ee['b2'], ee['g'], ee['be'])
    edges = _edge_enc(ps, pr, ee_w0, ee['W1'], ee['W2'], ee_bias)

    g0 = params['gnn'][0]['edge']['W0']
    nodes, S, R = _node_enc(x24, w0_eff, ne['W1'], ne['W2'], ne_bias,
                            g0[HID:2 * HID], g0[2 * HID:])

    zrows = jnp.zeros((N_PAD, HID), _F32)
    pred128 = None
    for l in range(LAYERS):
        lw = params['gnn'][l]
        ew = lw['edge']
        e_bias = _pack_bias(ew['b0'], ew['b1'], ew['b2'], ew['g'], ew['be'])
        gs, gr = _gather_pair(S, R, snd, rcv)
        edges = _edge_update(edges, gs, gr, ew['W0'][:HID], ew['W1'], ew['W2'], e_bias)
        aggf = _segment_sum(edges, rcv, zrows)
        agg = aggf.reshape(NC, N_PAD, HID)
        nw = lw['node']
        n_bias = _pack_bias(nw['b0'], nw['b1'], nw['b2'], nw['g'], nw['be'])
        if l < LAYERS - 1:
            gnext = params['gnn'][l + 1]['edge']['W0']
            nodes, S, R = _node_update(nodes, agg, nw['W0'][:HID], nw['W0'][HID:],
                                       nw['W1'], nw['W2'], n_bias,
                                       gnext[HID:2 * HID], gnext[2 * HID:])
        else:
            dec = params['dec']
            d2p = jnp.pad(dec['W2'], ((0, 0), (0, HID - 2)))
            db2p = jnp.pad(dec['b2'], (0, HID - 2))
            d_bias = _pack_bias(dec['b0'], dec['b1'], db2p)
            pred128 = _node_update_dec(nodes, agg, nw['W0'][:HID], nw['W0'][HID:],
                                       nw['W1'], nw['W2'], n_bias,
                                       dec['W0'], dec['W1'], d2p, d_bias)

    pred_acc = pred128[:, :2]

    pred_vel = poss[:, -1] - poss[:, -2]
    pred_pos = poss[:, -1] + pred_vel + pred_acc
    pred_pos = jnp.where(nonk_mask[:, None].astype(bool), pred_pos, tgt_poss[:, 0])
    pred_accns = pred_acc[:, None, :]
    pred_poss = pred_pos[:, None, :]
    return pred_accns, pred_poss


# speedup vs baseline: 1.5278x; 1.5278x over previous
import functools

import jax
import jax.numpy as jnp
from jax import lax
from jax.experimental import pallas as pl
from jax.experimental.pallas import tpu as pltpu
from jax.experimental.pallas import tpu_sc as plsc

N_NODES = 10000
N_EDGES = 160000
N_HIS = 6
HID = 128
EMB = 16
NTYPES = 9
LAYERS = 5
RADIUS = 0.1
EPS = 1e-5

NC = 2
NS = 16
NW = NC * NS

CH = 128
E_PAD = 163840
PER_W = E_PAD // NW
NCHUNK = PER_W // CH

N_PAD = 10240
ROWS_PER_TILE = N_PAD // NS

_F32 = jnp.float32


def _sc_mesh():
    return plsc.VectorSubcoreMesh(
        core_axis_name="c", subcore_axis_name="s", num_cores=NC, num_subcores=NS
    )


def _gather_pair(tab_a, tab_b, idx_a, idx_b):
    d = tab_a.shape[-1]

    def body(ta, tb, ia, ib, oa, ob, iva, ivb, rva, rvb, sema, semb):
        c = lax.axis_index("c")
        s = lax.axis_index("s")
        base = (s * NC + c) * PER_W

        def step(i, carry):
            off = base + i * CH
            pltpu.sync_copy(ia.at[pl.ds(off, CH)], iva)
            pltpu.sync_copy(ib.at[pl.ds(off, CH)], ivb)
            cpa = pltpu.async_copy(ta.at[iva], rva, sema)
            cpb = pltpu.async_copy(tb.at[ivb], rvb, semb)
            cpa.wait()
            cpb.wait()
            pltpu.sync_copy(rva, oa.at[pl.ds(off, CH)])
            pltpu.sync_copy(rvb, ob.at[pl.ds(off, CH)])
            return carry

        lax.fori_loop(0, NCHUNK, step, 0)

    f = pl.kernel(
        body,
        out_type=(
            jax.ShapeDtypeStruct((E_PAD, d), _F32),
            jax.ShapeDtypeStruct((E_PAD, d), _F32),
        ),
        mesh=_sc_mesh(),
        scratch_types=[
            pltpu.VMEM((CH,), jnp.int32),
            pltpu.VMEM((CH,), jnp.int32),
            pltpu.VMEM((CH, d), _F32),
            pltpu.VMEM((CH, d), _F32),
            pltpu.SemaphoreType.DMA,
            pltpu.SemaphoreType.DMA,
        ],
    )
    return f(tab_a, tab_b, idx_a, idx_b)


def _segment_sum(vals, idx, zeros_init):
    def body(vh, ih, zh, oh, iv, rv, acc):
        c = lax.axis_index("c")
        s = lax.axis_index("s")
        r0 = s * ROWS_PER_TILE
        pltpu.sync_copy(zh.at[pl.ds(r0, ROWS_PER_TILE)], acc.at[pl.ds(r0, ROWS_PER_TILE)])
        plsc.subcore_barrier()

        base = (s * NC + c) * PER_W

        def step(i, carry):
            off = base + i * CH
            pltpu.sync_copy(ih.at[pl.ds(off, CH)], iv)
            pltpu.sync_copy(vh.at[pl.ds(off, CH)], rv)
            pltpu.sync_copy(rv, acc.at[iv], add=True)
            return carry

        lax.fori_loop(0, NCHUNK, step, 0)
        plsc.subcore_barrier()
        pltpu.sync_copy(
            acc.at[pl.ds(r0, ROWS_PER_TILE)],
            oh.at[pl.ds(c * N_PAD + r0, ROWS_PER_TILE)],
        )

    f = pl.kernel(
        body,
        out_type=jax.ShapeDtypeStruct((NC * N_PAD, HID), _F32),
        mesh=_sc_mesh(),
        scratch_types=[
            pltpu.VMEM((CH,), jnp.int32),
            pltpu.VMEM((CH, HID), _F32),
            pltpu.VMEM_SHARED((N_PAD, HID), _F32),
        ],
    )
    return f(vals, idx, zeros_init)


def _dot(a, b):
    return jnp.dot(a, b, preferred_element_type=_F32,
                   precision=lax.Precision.HIGHEST)


def _mlp_tail(h, w1, w2, b1, b2, g, be):
    h = jnp.maximum(h, 0.0)
    h = jnp.maximum(_dot(h, w1) + b1, 0.0)
    h = _dot(h, w2) + b2
    mu = jnp.mean(h, axis=-1, keepdims=True)
    var = jnp.mean((h - mu) * (h - mu), axis=-1, keepdims=True)
    return (h - mu) * lax.rsqrt(var + EPS) * g + be


def _pack_bias(*rows):
    mat = jnp.stack(list(rows) + [jnp.zeros((HID,), _F32)] * (8 - len(rows)))
    return mat


_BE = 2048
_BN = 1000


def _full(shape):
    return pl.BlockSpec(shape, lambda i: tuple(0 for _ in shape))


def _rows(shape):
    return pl.BlockSpec(shape, lambda i: (i,) + tuple(0 for _ in shape[1:]))


def _edge_enc(ps, pr, w0p, w1, w2, bias):
    def body(ps_ref, pr_ref, w0_ref, w1_ref, w2_ref, bb_ref, o_ref):
        dx = (ps_ref[:, 0:1] - pr_ref[:, 0:1]) / RADIUS
        dy = (ps_ref[:, 1:2] - pr_ref[:, 1:2]) / RADIUS
        dist = jnp.sqrt(dx * dx + dy * dy)
        w0 = w0_ref[...]
        bb = bb_ref[...]
        h = dx * w0[0:1] + dy * w0[1:2] + dist * w0[2:3] + bb[0:1]
        o_ref[...] = _mlp_tail(h, w1_ref[...], w2_ref[...], bb[1:2], bb[2:3],
                               bb[3:4], bb[4:5])

    return pl.pallas_call(
        body,
        grid=(E_PAD // _BE,),
        in_specs=[
            _rows((_BE, HID)),
            _rows((_BE, HID)),
            _full((8, HID)),
            _full((HID, HID)),
            _full((HID, HID)),
            _full((8, HID)),
        ],
        out_specs=_rows((_BE, HID)),
        out_shape=jax.ShapeDtypeStruct((E_PAD, HID), _F32),
    )(ps, pr, w0p, w1, w2, bias)


def _node_enc(x, w0p, w1, w2, bias, ws_next, wr_next):
    def body(x_ref, w0_ref, w1_ref, w2_ref, bb_ref, ws_ref, wr_ref,
             on_ref, os_ref, or_ref):
        bb = bb_ref[...]
        h = _dot(x_ref[...], w0_ref[...]) + bb[0:1]
        y = _mlp_tail(h, w1_ref[...], w2_ref[...], bb[1:2], bb[2:3],
                      bb[3:4], bb[4:5])
        on_ref[...] = y
        os_ref[...] = _dot(y, ws_ref[...])
        or_ref[...] = _dot(y, wr_ref[...])

    sds = jax.ShapeDtypeStruct((N_NODES, HID), _F32)
    return pl.pallas_call(
        body,
        grid=(N_NODES // _BN,),
        in_specs=[
            _rows((_BN, 24)),
            _full((24, HID)),
            _full((HID, HID)),
            _full((HID, HID)),
            _full((8, HID)),
            _full((HID, HID)),
            _full((HID, HID)),
        ],
        out_specs=[_rows((_BN, HID))] * 3,
        out_shape=[sds, sds, sds],
    )(x, w0p, w1, w2, bias, ws_next, wr_next)


def _edge_update(edges, gs, gr, we, w1, w2, bias):
    def body(e_ref, gs_ref, gr_ref, we_ref, w1_ref, w2_ref, bb_ref, o_ref):
        bb = bb_ref[...]
        x = e_ref[...]
        h = _dot(x, we_ref[...]) + gs_ref[...] + gr_ref[...] + bb[0:1]
        y = _mlp_tail(h, w1_ref[...], w2_ref[...], bb[1:2], bb[2:3],
                      bb[3:4], bb[4:5])
        out = x + y
        row = lax.broadcasted_iota(jnp.int32, (_BE, 1), 0) + pl.program_id(0) * _BE
        o_ref[...] = jnp.where(row < N_EDGES, out, 0.0)

    return pl.pallas_call(
        body,
        grid=(E_PAD // _BE,),
        in_specs=[
            _rows((_BE, HID)),
            _rows((_BE, HID)),
            _rows((_BE, HID)),
            _full((HID, HID)),
            _full((HID, HID)),
            _full((HID, HID)),
            _full((8, HID)),
        ],
        out_specs=_rows((_BE, HID)),
        out_shape=jax.ShapeDtypeStruct((E_PAD, HID), _F32),
    )(edges, gs, gr, we, w1, w2, bias)


def _node_update(nodes, agg, wn, wa, w1, w2, bias, ws_next, wr_next):
    def body(n_ref, a_ref, wn_ref, wa_ref, w1_ref, w2_ref, bb_ref,
             ws_ref, wr_ref, on_ref, os_ref, or_ref):
        bb = bb_ref[...]
        x = n_ref[...]
        a = a_ref[0] + a_ref[1]
        h = _dot(x, wn_ref[...]) + _dot(a, wa_ref[...]) + bb[0:1]
        y = _mlp_tail(h, w1_ref[...], w2_ref[...], bb[1:2], bb[2:3],
                      bb[3:4], bb[4:5])
        nn = x + y
        on_ref[...] = nn
        os_ref[...] = _dot(nn, ws_ref[...])
        or_ref[...] = _dot(nn, wr_ref[...])

    sds = jax.ShapeDtypeStruct((N_NODES, HID), _F32)
    return pl.pallas_call(
        body,
        grid=(N_NODES // _BN,),
        in_specs=[
            _rows((_BN, HID)),
            pl.BlockSpec((NC, _BN, HID), lambda i: (0, i, 0)),
            _full((HID, HID)),
            _full((HID, HID)),
            _full((HID, HID)),
            _full((HID, HID)),
            _full((8, HID)),
            _full((HID, HID)),
            _full((HID, HID)),
        ],
        out_specs=[_rows((_BN, HID))] * 3,
        out_shape=[sds, sds, sds],
    )(nodes, agg, wn, wa, w1, w2, bias, ws_next, wr_next)


def _node_update_dec(nodes, agg, wn, wa, w1, w2, bias, d0, d1, d2p, dbias):
    def body(n_ref, a_ref, wn_ref, wa_ref, w1_ref, w2_ref, bb_ref,
             d0_ref, d1_ref, d2_ref, db_ref, o_ref):
        bb = bb_ref[...]
        db = db_ref[...]
        x = n_ref[...]
        a = a_ref[0] + a_ref[1]
        h = _dot(x, wn_ref[...]) + _dot(a, wa_ref[...]) + bb[0:1]
        y = _mlp_tail(h, w1_ref[...], w2_ref[...], bb[1:2], bb[2:3],
                      bb[3:4], bb[4:5])
        nn = x + y
        h = jnp.maximum(_dot(nn, d0_ref[...]) + db[0:1], 0.0)
        h = jnp.maximum(_dot(h, d1_ref[...]) + db[1:2], 0.0)
        o_ref[...] = _dot(h, d2_ref[...]) + db[2:3]

    return pl.pallas_call(
        body,
        grid=(N_NODES // _BN,),
        in_specs=[
            _rows((_BN, HID)),
            pl.BlockSpec((NC, _BN, HID), lambda i: (0, i, 0)),
            _full((HID, HID)),
            _full((HID, HID)),
            _full((HID, HID)),
            _full((HID, HID)),
            _full((8, HID)),
            _full((HID, HID)),
            _full((HID, HID)),
            _full((HID, HID)),
            _full((8, HID)),
        ],
        out_specs=_rows((_BN, HID)),
        out_shape=jax.ShapeDtypeStruct((N_NODES, HID), _F32),
    )(nodes, agg, wn, wa, w1, w2, bias, d0, d1, d2p, dbias)


def kernel(poss, particle_type, bounds, nonk_mask, tgt_poss, senders,
           receivers, params, num_rollouts):
    poss = poss + (jnp.asarray(num_rollouts) * 0).astype(poss.dtype)
    pos_last = poss[:, -1]

    vels = (poss[:, 1:] - poss[:, :-1]).reshape(N_NODES, (N_HIS - 1) * 2)
    d2w = jnp.concatenate([pos_last - bounds[:, 0], -pos_last + bounds[:, 1]], axis=1)
    d2w = jnp.clip(d2w / RADIUS, -1.0, 1.0)
    onehot = (particle_type[:, None] == jnp.arange(NTYPES)[None, :]).astype(_F32)
    x24 = jnp.concatenate([vels, d2w, onehot, jnp.zeros((N_NODES, 1), _F32)], axis=1)

    ne = params['node_enc']
    w0_eff = jnp.concatenate(
        [ne['W0'][:14], params['emb'] @ ne['W0'][14:30], jnp.zeros((1, HID), _F32)], axis=0)
    ne_bias = _pack_bias(ne['b0'], ne['b1'], ne['b2'], ne['g'], ne['be'])

    pad = jnp.zeros((E_PAD - N_EDGES,), jnp.int32)
    snd = jnp.concatenate([senders.astype(jnp.int32), pad])
    rcv = jnp.concatenate([receivers.astype(jnp.int32), pad])

    pos128 = jnp.pad(pos_last, ((0, 0), (0, HID - 2)))
    ps, pr = _gather_pair(pos128, pos128, snd, rcv)

    ee = params['edge_enc']
    ee_w0 = jnp.concatenate([ee['W0'], jnp.zeros((5, HID), _F32)], axis=0)
    ee_bias = _pack_bias(ee['b0'], ee['b1'], ee['b2'], ee['g'], ee['be'])
    edges = _edge_enc(ps, pr, ee_w0, ee['W1'], ee['W2'], ee_bias)

    g0 = params['gnn'][0]['edge']['W0']
    nodes, S, R = _node_enc(x24, w0_eff, ne['W1'], ne['W2'], ne_bias,
                            g0[HID:2 * HID], g0[2 * HID:])

    zrows = jnp.zeros((N_PAD, HID), _F32)
    pred128 = None
    for l in range(LAYERS):
        lw = params['gnn'][l]
        ew = lw['edge']
        e_bias = _pack_bias(ew['b0'], ew['b1'], ew['b2'], ew['g'], ew['be'])
        gs, gr = _gather_pair(S, R, snd, rcv)
        edges = _edge_update(edges, gs, gr, ew['W0'][:HID], ew['W1'], ew['W2'], e_bias)
        aggf = _segment_sum(edges, rcv, zrows)
        agg = aggf.reshape(NC, N_PAD, HID)
        nw = lw['node']
        n_bias = _pack_bias(nw['b0'], nw['b1'], nw['b2'], nw['g'], nw['be'])
        if l < LAYERS - 1:
            gnext = params['gnn'][l + 1]['edge']['W0']
            nodes, S, R = _node_update(nodes, agg, nw['W0'][:HID], nw['W0'][HID:],
                                       nw['W1'], nw['W2'], n_bias,
                                       gnext[HID:2 * HID], gnext[2 * HID:])
        else:
            dec = params['dec']
            d2p = jnp.pad(dec['W2'], ((0, 0), (0, HID - 2)))
            db2p = jnp.pad(dec['b2'], (0, HID - 2))
            d_bias = _pack_bias(dec['b0'], dec['b1'], db2p)
            pred128 = _node_update_dec(nodes, agg, nw['W0'][:HID], nw['W0'][HID:],
                                       nw['W1'], nw['W2'], n_bias,
                                       dec['W0'], dec['W1'], d2p, d_bias)

    pred_acc = pred128[:, :2]

    pred_vel = poss[:, -1] - poss[:, -2]
    pred_pos = poss[:, -1] + pred_vel + pred_acc
    pred_pos = jnp.where(nonk_mask[:, None].astype(bool), pred_pos, tgt_poss[:, 0])
    pred_accns = pred_acc[:, None, :]
    pred_poss = pred_pos[:, None, :]
    return pred_accns, pred_poss


# speedup vs baseline: 1.7794x; 1.1647x over previous
import functools

import jax
import jax.numpy as jnp
from jax import lax
from jax.experimental import pallas as pl
from jax.experimental.pallas import tpu as pltpu
from jax.experimental.pallas import tpu_sc as plsc

N_NODES = 10000
N_EDGES = 160000
N_HIS = 6
HID = 128
EMB = 16
NTYPES = 9
LAYERS = 5
RADIUS = 0.1
EPS = 1e-5

NC = 2
NS = 16
NW = NC * NS

CH = 128
E_PAD = 163840
PER_W = E_PAD // NW
NCHUNK = PER_W // CH

N_PAD = 10240
ROWS_PER_TILE = N_PAD // NS

_F32 = jnp.float32


def _sc_mesh():
    return plsc.VectorSubcoreMesh(
        core_axis_name="c", subcore_axis_name="s", num_cores=NC, num_subcores=NS
    )


def _gather_pair(tab_a, tab_b, idx_a, idx_b):
    d = tab_a.shape[-1]

    def body(ta, tb, ia, ib, oa, ob, iva, ivb,
             rva0, rva1, rvb0, rvb1, sa0, sa1, sb0, sb1):
        c = lax.axis_index("c")
        s = lax.axis_index("s")
        base = (s * NC + c) * PER_W
        rva = (rva0, rva1)
        rvb = (rvb0, rvb1)
        sa = (sa0, sa1)
        sb = (sb0, sb1)
        pltpu.sync_copy(ia.at[pl.ds(base, PER_W)], iva)
        pltpu.sync_copy(ib.at[pl.ds(base, PER_W)], ivb)

        def issue(j, p):
            pltpu.async_copy(ta.at[iva.at[pl.ds(j * CH, CH)]], rva[p], sa[p])
            pltpu.async_copy(tb.at[ivb.at[pl.ds(j * CH, CH)]], rvb[p], sb[p])

        def drain(j, p):
            pltpu.make_async_copy(ta, rva[p], sa[p]).wait()
            pltpu.make_async_copy(tb, rvb[p], sb[p]).wait()
            off = base + j * CH
            pltpu.sync_copy(rva[p], oa.at[pl.ds(off, CH)])
            pltpu.sync_copy(rvb[p], ob.at[pl.ds(off, CH)])

        issue(0, 0)
        issue(1, 1)

        def outer(i2, carry):
            for p in (0, 1):
                j = i2 * 2 + p
                drain(j, p)
                issue(j + 2, p)
            return carry

        lax.fori_loop(0, NCHUNK // 2 - 1, outer, 0)
        for p in (0, 1):
            drain(NCHUNK - 2 + p, p)

    f = pl.kernel(
        body,
        out_type=(
            jax.ShapeDtypeStruct((E_PAD, d), _F32),
            jax.ShapeDtypeStruct((E_PAD, d), _F32),
        ),
        mesh=_sc_mesh(),
        scratch_types=[
            pltpu.VMEM((PER_W,), jnp.int32),
            pltpu.VMEM((PER_W,), jnp.int32),
            pltpu.VMEM((CH, d), _F32),
            pltpu.VMEM((CH, d), _F32),
            pltpu.VMEM((CH, d), _F32),
            pltpu.VMEM((CH, d), _F32),
            pltpu.SemaphoreType.DMA,
            pltpu.SemaphoreType.DMA,
            pltpu.SemaphoreType.DMA,
            pltpu.SemaphoreType.DMA,
        ],
    )
    return f(tab_a, tab_b, idx_a, idx_b)


def _segment_sum(vals, idx, zeros_init):
    def body(vh, ih, zh, oh, iv0, iv1, rv0, rv1, sm0, sm1, acc):
        c = lax.axis_index("c")
        s = lax.axis_index("s")
        r0 = s * ROWS_PER_TILE
        pltpu.sync_copy(zh.at[pl.ds(r0, ROWS_PER_TILE)], acc.at[pl.ds(r0, ROWS_PER_TILE)])
        plsc.subcore_barrier()

        base = (s * NC + c) * PER_W
        iv = (iv0, iv1)
        rv = (rv0, rv1)
        sm = (sm0, sm1)

        def issue(j, p):
            off = base + j * CH
            pltpu.sync_copy(ih.at[pl.ds(off, CH)], iv[p])
            pltpu.async_copy(vh.at[pl.ds(off, CH)], rv[p], sm[p])

        def drain(p):
            pltpu.make_async_copy(vh.at[pl.ds(0, CH)], rv[p], sm[p]).wait()
            pltpu.sync_copy(rv[p], acc.at[iv[p]], add=True)

        issue(0, 0)
        issue(1, 1)

        def outer(i2, carry):
            for p in (0, 1):
                drain(p)
                issue(i2 * 2 + p + 2, p)
            return carry

        lax.fori_loop(0, NCHUNK // 2 - 1, outer, 0)
        for p in (0, 1):
            drain(p)
        plsc.subcore_barrier()
        pltpu.sync_copy(
            acc.at[pl.ds(r0, ROWS_PER_TILE)],
            oh.at[pl.ds(c * N_PAD + r0, ROWS_PER_TILE)],
        )

    f = pl.kernel(
        body,
        out_type=jax.ShapeDtypeStruct((NC * N_PAD, HID), _F32),
        mesh=_sc_mesh(),
        scratch_types=[
            pltpu.VMEM((CH,), jnp.int32),
            pltpu.VMEM((CH,), jnp.int32),
            pltpu.VMEM((CH, HID), _F32),
            pltpu.VMEM((CH, HID), _F32),
            pltpu.SemaphoreType.DMA,
            pltpu.SemaphoreType.DMA,
            pltpu.VMEM_SHARED((N_PAD, HID), _F32),
        ],
    )
    return f(vals, idx, zeros_init)


def _dot(a, b):
    return jnp.dot(a, b, preferred_element_type=_F32,
                   precision=lax.Precision.HIGHEST)


def _mlp_tail(h, w1, w2, b1, b2, g, be):
    h = jnp.maximum(h, 0.0)
    h = jnp.maximum(_dot(h, w1) + b1, 0.0)
    h = _dot(h, w2) + b2
    mu = jnp.mean(h, axis=-1, keepdims=True)
    var = jnp.mean((h - mu) * (h - mu), axis=-1, keepdims=True)
    return (h - mu) * lax.rsqrt(var + EPS) * g + be


def _pack_bias(*rows):
    mat = jnp.stack(list(rows) + [jnp.zeros((HID,), _F32)] * (8 - len(rows)))
    return mat


_BE = 2048
_BN = 1000


def _full(shape):
    return pl.BlockSpec(shape, lambda i: tuple(0 for _ in shape))


def _rows(shape):
    return pl.BlockSpec(shape, lambda i: (i,) + tuple(0 for _ in shape[1:]))


def _edge_enc(ps, pr, w0p, w1, w2, bias):
    def body(ps_ref, pr_ref, w0_ref, w1_ref, w2_ref, bb_ref, o_ref):
        dx = (ps_ref[:, 0:1] - pr_ref[:, 0:1]) * (1.0 / RADIUS)
        dy = (ps_ref[:, 1:2] - pr_ref[:, 1:2]) * (1.0 / RADIUS)
        dist = jnp.sqrt(dx * dx + dy * dy)
        w0 = w0_ref[...]
        bb = bb_ref[...]
        h = dx * w0[0:1] + dy * w0[1:2] + dist * w0[2:3] + bb[0:1]
        o_ref[...] = _mlp_tail(h, w1_ref[...], w2_ref[...], bb[1:2], bb[2:3],
                               bb[3:4], bb[4:5])

    return pl.pallas_call(
        body,
        grid=(E_PAD // _BE,),
        in_specs=[
            _rows((_BE, HID)),
            _rows((_BE, HID)),
            _full((8, HID)),
            _full((HID, HID)),
            _full((HID, HID)),
            _full((8, HID)),
        ],
        out_specs=_rows((_BE, HID)),
        out_shape=jax.ShapeDtypeStruct((E_PAD, HID), _F32),
    )(ps, pr, w0p, w1, w2, bias)


def _node_enc(x, w0p, w1, w2, bias, ws_next, wr_next):
    def body(x_ref, w0_ref, w1_ref, w2_ref, bb_ref, ws_ref, wr_ref,
             on_ref, os_ref, or_ref):
        bb = bb_ref[...]
        h = _dot(x_ref[...], w0_ref[...]) + bb[0:1]
        y = _mlp_tail(h, w1_ref[...], w2_ref[...], bb[1:2], bb[2:3],
                      bb[3:4], bb[4:5])
        on_ref[...] = y
        os_ref[...] = _dot(y, ws_ref[...])
        or_ref[...] = _dot(y, wr_ref[...])

    sds = jax.ShapeDtypeStruct((N_NODES, HID), _F32)
    return pl.pallas_call(
        body,
        grid=(N_NODES // _BN,),
        in_specs=[
            _rows((_BN, 24)),
            _full((24, HID)),
            _full((HID, HID)),
            _full((HID, HID)),
            _full((8, HID)),
            _full((HID, HID)),
            _full((HID, HID)),
        ],
        out_specs=[_rows((_BN, HID))] * 3,
        out_shape=[sds, sds, sds],
    )(x, w0p, w1, w2, bias, ws_next, wr_next)


def _edge_update(edges, gs, gr, we, w1, w2, bias):
    def body(e_ref, gs_ref, gr_ref, we_ref, w1_ref, w2_ref, bb_ref, o_ref):
        bb = bb_ref[...]
        x = e_ref[...]
        h = _dot(x, we_ref[...]) + gs_ref[...] + gr_ref[...] + bb[0:1]
        y = _mlp_tail(h, w1_ref[...], w2_ref[...], bb[1:2], bb[2:3],
                      bb[3:4], bb[4:5])
        out = x + y
        row = lax.broadcasted_iota(jnp.int32, (_BE, 1), 0) + pl.program_id(0) * _BE
        o_ref[...] = jnp.where(row < N_EDGES, out, 0.0)

    return pl.pallas_call(
        body,
        grid=(E_PAD // _BE,),
        in_specs=[
            _rows((_BE, HID)),
            _rows((_BE, HID)),
            _rows((_BE, HID)),
            _full((HID, HID)),
            _full((HID, HID)),
            _full((HID, HID)),
            _full((8, HID)),
        ],
        out_specs=_rows((_BE, HID)),
        out_shape=jax.ShapeDtypeStruct((E_PAD, HID), _F32),
    )(edges, gs, gr, we, w1, w2, bias)


def _node_update(nodes, agg, wn, wa, w1, w2, bias, ws_next, wr_next):
    def body(n_ref, a_ref, wn_ref, wa_ref, w1_ref, w2_ref, bb_ref,
             ws_ref, wr_ref, on_ref, os_ref, or_ref):
        bb = bb_ref[...]
        x = n_ref[...]
        a = a_ref[0] + a_ref[1]
        h = _dot(x, wn_ref[...]) + _dot(a, wa_ref[...]) + bb[0:1]
        y = _mlp_tail(h, w1_ref[...], w2_ref[...], bb[1:2], bb[2:3],
                      bb[3:4], bb[4:5])
        nn = x + y
        on_ref[...] = nn
        os_ref[...] = _dot(nn, ws_ref[...])
        or_ref[...] = _dot(nn, wr_ref[...])

    sds = jax.ShapeDtypeStruct((N_NODES, HID), _F32)
    return pl.pallas_call(
        body,
        grid=(N_NODES // _BN,),
        in_specs=[
            _rows((_BN, HID)),
            pl.BlockSpec((NC, _BN, HID), lambda i: (0, i, 0)),
            _full((HID, HID)),
            _full((HID, HID)),
            _full((HID, HID)),
            _full((HID, HID)),
            _full((8, HID)),
            _full((HID, HID)),
            _full((HID, HID)),
        ],
        out_specs=[_rows((_BN, HID))] * 3,
        out_shape=[sds, sds, sds],
    )(nodes, agg, wn, wa, w1, w2, bias, ws_next, wr_next)


def _node_update_dec(nodes, agg, wn, wa, w1, w2, bias, d0, d1, d2p, dbias):
    def body(n_ref, a_ref, wn_ref, wa_ref, w1_ref, w2_ref, bb_ref,
             d0_ref, d1_ref, d2_ref, db_ref, o_ref):
        bb = bb_ref[...]
        db = db_ref[...]
        x = n_ref[...]
        a = a_ref[0] + a_ref[1]
        h = _dot(x, wn_ref[...]) + _dot(a, wa_ref[...]) + bb[0:1]
        y = _mlp_tail(h, w1_ref[...], w2_ref[...], bb[1:2], bb[2:3],
                      bb[3:4], bb[4:5])
        nn = x + y
        h = jnp.maximum(_dot(nn, d0_ref[...]) + db[0:1], 0.0)
        h = jnp.maximum(_dot(h, d1_ref[...]) + db[1:2], 0.0)
        o_ref[...] = _dot(h, d2_ref[...]) + db[2:3]

    return pl.pallas_call(
        body,
        grid=(N_NODES // _BN,),
        in_specs=[
            _rows((_BN, HID)),
            pl.BlockSpec((NC, _BN, HID), lambda i: (0, i, 0)),
            _full((HID, HID)),
            _full((HID, HID)),
            _full((HID, HID)),
            _full((HID, HID)),
            _full((8, HID)),
            _full((HID, HID)),
            _full((HID, HID)),
            _full((HID, HID)),
            _full((8, HID)),
        ],
        out_specs=_rows((_BN, HID)),
        out_shape=jax.ShapeDtypeStruct((N_NODES, HID), _F32),
    )(nodes, agg, wn, wa, w1, w2, bias, d0, d1, d2p, dbias)


def kernel(poss, particle_type, bounds, nonk_mask, tgt_poss, senders,
           receivers, params, num_rollouts):
    poss = poss + (jnp.asarray(num_rollouts) * 0).astype(poss.dtype)
    pos_last = poss[:, -1]

    vels = (poss[:, 1:] - poss[:, :-1]).reshape(N_NODES, (N_HIS - 1) * 2)
    d2w = jnp.concatenate([pos_last - bounds[:, 0], -pos_last + bounds[:, 1]], axis=1)
    d2w = jnp.clip(d2w / RADIUS, -1.0, 1.0)
    onehot = (particle_type[:, None] == jnp.arange(NTYPES)[None, :]).astype(_F32)
    x24 = jnp.concatenate([vels, d2w, onehot, jnp.zeros((N_NODES, 1), _F32)], axis=1)

    ne = params['node_enc']
    w0_eff = jnp.concatenate(
        [ne['W0'][:14], params['emb'] @ ne['W0'][14:30], jnp.zeros((1, HID), _F32)], axis=0)
    ne_bias = _pack_bias(ne['b0'], ne['b1'], ne['b2'], ne['g'], ne['be'])

    pad = jnp.zeros((E_PAD - N_EDGES,), jnp.int32)
    snd = jnp.concatenate([senders.astype(jnp.int32), pad])
    rcv = jnp.concatenate([receivers.astype(jnp.int32), pad])

    pos128 = jnp.pad(pos_last, ((0, 0), (0, HID - 2)))
    ps, pr = _gather_pair(pos128, pos128, snd, rcv)

    ee = params['edge_enc']
    ee_w0 = jnp.concatenate([ee['W0'], jnp.zeros((5, HID), _F32)], axis=0)
    ee_bias = _pack_bias(ee['b0'], ee['b1'], ee['b2'], ee['g'], ee['be'])
    edges = _edge_enc(ps, pr, ee_w0, ee['W1'], ee['W2'], ee_bias)

    g0 = params['gnn'][0]['edge']['W0']
    nodes, S, R = _node_enc(x24, w0_eff, ne['W1'], ne['W2'], ne_bias,
                            g0[HID:2 * HID], g0[2 * HID:])

    zrows = jnp.zeros((N_PAD, HID), _F32)
    pred128 = None
    for l in range(LAYERS):
        lw = params['gnn'][l]
        ew = lw['edge']
        e_bias = _pack_bias(ew['b0'], ew['b1'], ew['b2'], ew['g'], ew['be'])
        gs, gr = _gather_pair(S, R, snd, rcv)
        edges = _edge_update(edges, gs, gr, ew['W0'][:HID], ew['W1'], ew['W2'], e_bias)
        aggf = _segment_sum(edges, rcv, zrows)
        agg = aggf.reshape(NC, N_PAD, HID)
        nw = lw['node']
        n_bias = _pack_bias(nw['b0'], nw['b1'], nw['b2'], nw['g'], nw['be'])
        if l < LAYERS - 1:
            gnext = params['gnn'][l + 1]['edge']['W0']
            nodes, S, R = _node_update(nodes, agg, nw['W0'][:HID], nw['W0'][HID:],
                                       nw['W1'], nw['W2'], n_bias,
                                       gnext[HID:2 * HID], gnext[2 * HID:])
        else:
            dec = params['dec']
            d2p = jnp.pad(dec['W2'], ((0, 0), (0, HID - 2)))
            db2p = jnp.pad(dec['b2'], (0, HID - 2))
            d_bias = _pack_bias(dec['b0'], dec['b1'], db2p)
            pred128 = _node_update_dec(nodes, agg, nw['W0'][:HID], nw['W0'][HID:],
                                       nw['W1'], nw['W2'], n_bias,
                                       dec['W0'], dec['W1'], d2p, d_bias)

    pred_acc = pred128[:, :2]

    pred_vel = poss[:, -1] - poss[:, -2]
    pred_pos = poss[:, -1] + pred_vel + pred_acc
    pred_pos = jnp.where(nonk_mask[:, None].astype(bool), pred_pos, tgt_poss[:, 0])
    pred_accns = pred_acc[:, None, :]
    pred_poss = pred_pos[:, None, :]
    return pred_accns, pred_poss


# speedup vs baseline: 2.0654x; 1.1607x over previous
import functools

import jax
import jax.numpy as jnp
from jax import lax
from jax.experimental import pallas as pl
from jax.experimental.pallas import tpu as pltpu
from jax.experimental.pallas import tpu_sc as plsc

N_NODES = 10000
N_EDGES = 160000
N_HIS = 6
HID = 128
EMB = 16
NTYPES = 9
LAYERS = 5
RADIUS = 0.1
EPS = 1e-5

NC = 2
NS = 16
NW = NC * NS

CH = 128
E_PAD = 163840
E_HALF = E_PAD // 2

N_PAD = 10240
ROWS_PER_TILE = N_PAD // NS

_F32 = jnp.float32


def _sc_mesh():
    return plsc.VectorSubcoreMesh(
        core_axis_name="c", subcore_axis_name="s", num_cores=NC, num_subcores=NS
    )


def _gather_pair(tab_a, tab_b, idx_a, idx_b):
    d = tab_a.shape[-1]
    n = idx_a.shape[0]
    per_w = n // NW
    nchunk = per_w // CH

    def body(ta, tb, ia, ib, oa, ob, iva, ivb,
             rva0, rva1, rvb0, rvb1, sa0, sa1, sb0, sb1):
        c = lax.axis_index("c")
        s = lax.axis_index("s")
        base = (s * NC + c) * per_w
        rva = (rva0, rva1)
        rvb = (rvb0, rvb1)
        sa = (sa0, sa1)
        sb = (sb0, sb1)
        pltpu.sync_copy(ia.at[pl.ds(base, per_w)], iva)
        pltpu.sync_copy(ib.at[pl.ds(base, per_w)], ivb)

        def issue(j, p):
            pltpu.async_copy(ta.at[iva.at[pl.ds(j * CH, CH)]], rva[p], sa[p])
            pltpu.async_copy(tb.at[ivb.at[pl.ds(j * CH, CH)]], rvb[p], sb[p])

        def drain(j, p):
            pltpu.make_async_copy(ta, rva[p], sa[p]).wait()
            pltpu.make_async_copy(tb, rvb[p], sb[p]).wait()
            off = base + j * CH
            pltpu.sync_copy(rva[p], oa.at[pl.ds(off, CH)])
            pltpu.sync_copy(rvb[p], ob.at[pl.ds(off, CH)])

        issue(0, 0)
        issue(1, 1)

        def outer(i2, carry):
            for p in (0, 1):
                j = i2 * 2 + p
                drain(j, p)
                issue(j + 2, p)
            return carry

        lax.fori_loop(0, nchunk // 2 - 1, outer, 0)
        for p in (0, 1):
            drain(nchunk - 2 + p, p)

    f = pl.kernel(
        body,
        out_type=(
            jax.ShapeDtypeStruct((n, d), _F32),
            jax.ShapeDtypeStruct((n, d), _F32),
        ),
        mesh=_sc_mesh(),
        scratch_types=[
            pltpu.VMEM((per_w,), jnp.int32),
            pltpu.VMEM((per_w,), jnp.int32),
            pltpu.VMEM((CH, d), _F32),
            pltpu.VMEM((CH, d), _F32),
            pltpu.VMEM((CH, d), _F32),
            pltpu.VMEM((CH, d), _F32),
            pltpu.SemaphoreType.DMA,
            pltpu.SemaphoreType.DMA,
            pltpu.SemaphoreType.DMA,
            pltpu.SemaphoreType.DMA,
        ],
    )
    return f(tab_a, tab_b, idx_a, idx_b)


def _segment_sum(vals1, vals2, idx, zeros_init):
    per_h = E_HALF // NW
    nch_h = per_h // CH

    def body(vh1, vh2, ih, zh, oh, iv0, iv1, rv0, rv1, sm0, sm1, acc):
        c = lax.axis_index("c")
        s = lax.axis_index("s")
        r0 = s * ROWS_PER_TILE
        pltpu.sync_copy(zh.at[pl.ds(r0, ROWS_PER_TILE)], acc.at[pl.ds(r0, ROWS_PER_TILE)])
        plsc.subcore_barrier()

        base = (s * NC + c) * per_h
        vh = (vh1, vh2)
        iv = (iv0, iv1)
        rv = (rv0, rv1)
        sm = (sm0, sm1)

        def issue(j, p):
            off = base + j * CH
            pltpu.sync_copy(ih.at[pl.ds(off + p * E_HALF, CH)], iv[p])
            pltpu.async_copy(vh[p].at[pl.ds(off, CH)], rv[p], sm[p])

        def drain(p):
            pltpu.make_async_copy(vh[p].at[pl.ds(0, CH)], rv[p], sm[p]).wait()
            pltpu.sync_copy(rv[p], acc.at[iv[p]], add=True)

        issue(0, 0)
        issue(0, 1)

        def outer(i2, carry):
            for p in (0, 1):
                drain(p)
                issue(i2 + 1, p)
            return carry

        lax.fori_loop(0, nch_h - 1, outer, 0)
        for p in (0, 1):
            drain(p)
        plsc.subcore_barrier()
        pltpu.sync_copy(
            acc.at[pl.ds(r0, ROWS_PER_TILE)],
            oh.at[pl.ds(c * N_PAD + r0, ROWS_PER_TILE)],
        )

    f = pl.kernel(
        body,
        out_type=jax.ShapeDtypeStruct((NC * N_PAD, HID), _F32),
        mesh=_sc_mesh(),
        scratch_types=[
            pltpu.VMEM((CH,), jnp.int32),
            pltpu.VMEM((CH,), jnp.int32),
            pltpu.VMEM((CH, HID), _F32),
            pltpu.VMEM((CH, HID), _F32),
            pltpu.SemaphoreType.DMA,
            pltpu.SemaphoreType.DMA,
            pltpu.VMEM_SHARED((N_PAD, HID), _F32),
        ],
    )
    return f(vals1, vals2, idx, zeros_init)


def _dot(a, b):
    return jnp.dot(a, b, preferred_element_type=_F32,
                   precision=lax.Precision.HIGHEST)


def _mlp_tail(h, w1, w2, b1, b2, g, be):
    h = jnp.maximum(h, 0.0)
    h = jnp.maximum(_dot(h, w1) + b1, 0.0)
    h = _dot(h, w2) + b2
    mu = jnp.mean(h, axis=-1, keepdims=True)
    var = jnp.mean((h - mu) * (h - mu), axis=-1, keepdims=True)
    return (h - mu) * lax.rsqrt(var + EPS) * g + be


def _pack_bias(*rows):
    mat = jnp.stack(list(rows) + [jnp.zeros((HID,), _F32)] * (8 - len(rows)))
    return mat


_BE = 2048
_BN = 1000


def _full(shape):
    return pl.BlockSpec(shape, lambda i: tuple(0 for _ in shape))


def _rows(shape):
    return pl.BlockSpec(shape, lambda i: (i,) + tuple(0 for _ in shape[1:]))


def _edge_enc(ps, pr, w0p, w1, w2, bias):
    def body(ps_ref, pr_ref, w0_ref, w1_ref, w2_ref, bb_ref, o_ref):
        dx = (ps_ref[:, 0:1] - pr_ref[:, 0:1]) * (1.0 / RADIUS)
        dy = (ps_ref[:, 1:2] - pr_ref[:, 1:2]) * (1.0 / RADIUS)
        dist = jnp.sqrt(dx * dx + dy * dy)
        w0 = w0_ref[...]
        bb = bb_ref[...]
        h = dx * w0[0:1] + dy * w0[1:2] + dist * w0[2:3] + bb[0:1]
        o_ref[...] = _mlp_tail(h, w1_ref[...], w2_ref[...], bb[1:2], bb[2:3],
                               bb[3:4], bb[4:5])

    return pl.pallas_call(
        body,
        grid=(E_HALF // _BE,),
        in_specs=[
            _rows((_BE, HID)),
            _rows((_BE, HID)),
            _full((8, HID)),
            _full((HID, HID)),
            _full((HID, HID)),
            _full((8, HID)),
        ],
        out_specs=_rows((_BE, HID)),
        out_shape=jax.ShapeDtypeStruct((E_HALF, HID), _F32),
    )(ps, pr, w0p, w1, w2, bias)


def _node_enc(x, w0p, w1, w2, bias, ws_next, wr_next):
    def body(x_ref, w0_ref, w1_ref, w2_ref, bb_ref, ws_ref, wr_ref,
             on_ref, os_ref, or_ref):
        bb = bb_ref[...]
        h = _dot(x_ref[...], w0_ref[...]) + bb[0:1]
        y = _mlp_tail(h, w1_ref[...], w2_ref[...], bb[1:2], bb[2:3],
                      bb[3:4], bb[4:5])
        on_ref[...] = y
        os_ref[...] = _dot(y, ws_ref[...])
        or_ref[...] = _dot(y, wr_ref[...])

    sds = jax.ShapeDtypeStruct((N_NODES, HID), _F32)
    return pl.pallas_call(
        body,
        grid=(N_NODES // _BN,),
        in_specs=[
            _rows((_BN, 24)),
            _full((24, HID)),
            _full((HID, HID)),
            _full((HID, HID)),
            _full((8, HID)),
            _full((HID, HID)),
            _full((HID, HID)),
        ],
        out_specs=[_rows((_BN, HID))] * 3,
        out_shape=[sds, sds, sds],
    )(x, w0p, w1, w2, bias, ws_next, wr_next)


def _edge_update(edges, gs, gr, we, w1, w2, bias, row0):
    def body(e_ref, gs_ref, gr_ref, we_ref, w1_ref, w2_ref, bb_ref, o_ref):
        bb = bb_ref[...]
        x = e_ref[...]
        h = _dot(x, we_ref[...]) + gs_ref[...] + gr_ref[...] + bb[0:1]
        y = _mlp_tail(h, w1_ref[...], w2_ref[...], bb[1:2], bb[2:3],
                      bb[3:4], bb[4:5])
        out = x + y
        row = lax.broadcasted_iota(jnp.int32, (_BE, 1), 0) + (
            row0 + pl.program_id(0) * _BE)
        o_ref[...] = jnp.where(row < N_EDGES, out, 0.0)

    return pl.pallas_call(
        body,
        grid=(E_HALF // _BE,),
        in_specs=[
            _rows((_BE, HID)),
            _rows((_BE, HID)),
            _rows((_BE, HID)),
            _full((HID, HID)),
            _full((HID, HID)),
            _full((HID, HID)),
            _full((8, HID)),
        ],
        out_specs=_rows((_BE, HID)),
        out_shape=jax.ShapeDtypeStruct((E_HALF, HID), _F32),
    )(edges, gs, gr, we, w1, w2, bias)


def _node_update(nodes, agg, wn, wa, w1, w2, bias, ws_next, wr_next):
    def body(n_ref, a_ref, wn_ref, wa_ref, w1_ref, w2_ref, bb_ref,
             ws_ref, wr_ref, on_ref, os_ref, or_ref):
        bb = bb_ref[...]
        x = n_ref[...]
        a = a_ref[0] + a_ref[1]
        h = _dot(x, wn_ref[...]) + _dot(a, wa_ref[...]) + bb[0:1]
        y = _mlp_tail(h, w1_ref[...], w2_ref[...], bb[1:2], bb[2:3],
                      bb[3:4], bb[4:5])
        nn = x + y
        on_ref[...] = nn
        os_ref[...] = _dot(nn, ws_ref[...])
        or_ref[...] = _dot(nn, wr_ref[...])

    sds = jax.ShapeDtypeStruct((N_NODES, HID), _F32)
    return pl.pallas_call(
        body,
        grid=(N_NODES // _BN,),
        in_specs=[
            _rows((_BN, HID)),
            pl.BlockSpec((NC, _BN, HID), lambda i: (0, i, 0)),
            _full((HID, HID)),
            _full((HID, HID)),
            _full((HID, HID)),
            _full((HID, HID)),
            _full((8, HID)),
            _full((HID, HID)),
            _full((HID, HID)),
        ],
        out_specs=[_rows((_BN, HID))] * 3,
        out_shape=[sds, sds, sds],
    )(nodes, agg, wn, wa, w1, w2, bias, ws_next, wr_next)


def _node_update_dec(nodes, agg, wn, wa, w1, w2, bias, d0, d1, d2p, dbias):
    def body(n_ref, a_ref, wn_ref, wa_ref, w1_ref, w2_ref, bb_ref,
             d0_ref, d1_ref, d2_ref, db_ref, o_ref):
        bb = bb_ref[...]
        db = db_ref[...]
        x = n_ref[...]
        a = a_ref[0] + a_ref[1]
        h = _dot(x, wn_ref[...]) + _dot(a, wa_ref[...]) + bb[0:1]
        y = _mlp_tail(h, w1_ref[...], w2_ref[...], bb[1:2], bb[2:3],
                      bb[3:4], bb[4:5])
        nn = x + y
        h = jnp.maximum(_dot(nn, d0_ref[...]) + db[0:1], 0.0)
        h = jnp.maximum(_dot(h, d1_ref[...]) + db[1:2], 0.0)
        o_ref[...] = _dot(h, d2_ref[...]) + db[2:3]

    return pl.pallas_call(
        body,
        grid=(N_NODES // _BN,),
        in_specs=[
            _rows((_BN, HID)),
            pl.BlockSpec((NC, _BN, HID), lambda i: (0, i, 0)),
            _full((HID, HID)),
            _full((HID, HID)),
            _full((HID, HID)),
            _full((HID, HID)),
            _full((8, HID)),
            _full((HID, HID)),
            _full((HID, HID)),
            _full((HID, HID)),
            _full((8, HID)),
        ],
        out_specs=_rows((_BN, HID)),
        out_shape=jax.ShapeDtypeStruct((N_NODES, HID), _F32),
    )(nodes, agg, wn, wa, w1, w2, bias, d0, d1, d2p, dbias)


def kernel(poss, particle_type, bounds, nonk_mask, tgt_poss, senders,
           receivers, params, num_rollouts):
    poss = poss + (jnp.asarray(num_rollouts) * 0).astype(poss.dtype)
    pos_last = poss[:, -1]

    vels = (poss[:, 1:] - poss[:, :-1]).reshape(N_NODES, (N_HIS - 1) * 2)
    d2w = jnp.concatenate([pos_last - bounds[:, 0], -pos_last + bounds[:, 1]], axis=1)
    d2w = jnp.clip(d2w / RADIUS, -1.0, 1.0)
    onehot = (particle_type[:, None] == jnp.arange(NTYPES)[None, :]).astype(_F32)
    x24 = jnp.concatenate([vels, d2w, onehot, jnp.zeros((N_NODES, 1), _F32)], axis=1)

    ne = params['node_enc']
    w0_eff = jnp.concatenate(
        [ne['W0'][:14], params['emb'] @ ne['W0'][14:30], jnp.zeros((1, HID), _F32)], axis=0)
    ne_bias = _pack_bias(ne['b0'], ne['b1'], ne['b2'], ne['g'], ne['be'])

    pad = jnp.zeros((E_PAD - N_EDGES,), jnp.int32)
    snd = jnp.concatenate([senders.astype(jnp.int32), pad])
    rcv = jnp.concatenate([receivers.astype(jnp.int32), pad])
    snd1, snd2 = snd[:E_HALF], snd[E_HALF:]
    rcv1, rcv2 = rcv[:E_HALF], rcv[E_HALF:]

    pos128 = jnp.pad(pos_last, ((0, 0), (0, HID - 2)))
    ps1, pr1 = _gather_pair(pos128, pos128, snd1, rcv1)
    ps2, pr2 = _gather_pair(pos128, pos128, snd2, rcv2)

    ee = params['edge_enc']
    ee_w0 = jnp.concatenate([ee['W0'], jnp.zeros((5, HID), _F32)], axis=0)
    ee_bias = _pack_bias(ee['b0'], ee['b1'], ee['b2'], ee['g'], ee['be'])
    edges1 = _edge_enc(ps1, pr1, ee_w0, ee['W1'], ee['W2'], ee_bias)
    edges2 = _edge_enc(ps2, pr2, ee_w0, ee['W1'], ee['W2'], ee_bias)

    g0 = params['gnn'][0]['edge']['W0']
    nodes, S, R = _node_enc(x24, w0_eff, ne['W1'], ne['W2'], ne_bias,
                            g0[HID:2 * HID], g0[2 * HID:])

    zrows = jnp.zeros((N_PAD, HID), _F32)
    pred128 = None
    for l in range(LAYERS):
        lw = params['gnn'][l]
        ew = lw['edge']
        e_bias = _pack_bias(ew['b0'], ew['b1'], ew['b2'], ew['g'], ew['be'])
        gs1, gr1 = _gather_pair(S, R, snd1, rcv1)
        gs2, gr2 = _gather_pair(S, R, snd2, rcv2)
        edges1 = _edge_update(edges1, gs1, gr1, ew['W0'][:HID], ew['W1'],
                              ew['W2'], e_bias, 0)
        edges2 = _edge_update(edges2, gs2, gr2, ew['W0'][:HID], ew['W1'],
                              ew['W2'], e_bias, E_HALF)
        aggf = _segment_sum(edges1, edges2, rcv, zrows)
        agg = aggf.reshape(NC, N_PAD, HID)
        nw = lw['node']
        n_bias = _pack_bias(nw['b0'], nw['b1'], nw['b2'], nw['g'], nw['be'])
        if l < LAYERS - 1:
            gnext = params['gnn'][l + 1]['edge']['W0']
            nodes, S, R = _node_update(nodes, agg, nw['W0'][:HID], nw['W0'][HID:],
                                       nw['W1'], nw['W2'], n_bias,
                                       gnext[HID:2 * HID], gnext[2 * HID:])
        else:
            dec = params['dec']
            d2p = jnp.pad(dec['W2'], ((0, 0), (0, HID - 2)))
            db2p = jnp.pad(dec['b2'], (0, HID - 2))
            d_bias = _pack_bias(dec['b0'], dec['b1'], db2p)
            pred128 = _node_update_dec(nodes, agg, nw['W0'][:HID], nw['W0'][HID:],
                                       nw['W1'], nw['W2'], n_bias,
                                       dec['W0'], dec['W1'], d2p, d_bias)

    pred_acc = pred128[:, :2]

    pred_vel = poss[:, -1] - poss[:, -2]
    pred_pos = poss[:, -1] + pred_vel + pred_acc
    pred_pos = jnp.where(nonk_mask[:, None].astype(bool), pred_pos, tgt_poss[:, 0])
    pred_accns = pred_acc[:, None, :]
    pred_poss = pred_pos[:, None, :]
    return pred_accns, pred_poss


# speedup vs baseline: 2.4590x; 1.1906x over previous
import functools

import jax
import jax.numpy as jnp
from jax import lax
from jax.experimental import pallas as pl
from jax.experimental.pallas import tpu as pltpu
from jax.experimental.pallas import tpu_sc as plsc

N_NODES = 10000
N_EDGES = 160000
N_HIS = 6
HID = 128
EMB = 16
NTYPES = 9
LAYERS = 5
RADIUS = 0.1
EPS = 1e-5

NC = 2
NS = 16
NW = NC * NS

CH = 128
E_PAD = 163840
E_HALF = E_PAD // 2

N_PAD = 10240
ROWS_PER_TILE = N_PAD // NS

_F32 = jnp.float32


def _sc_mesh():
    return plsc.VectorSubcoreMesh(
        core_axis_name="c", subcore_axis_name="s", num_cores=NC, num_subcores=NS
    )


def _gather_pair(tab_a, tab_b, idx_a, idx_b):
    d = tab_a.shape[-1]
    n = idx_a.shape[0]
    per_w = n // NW
    nchunk = per_w // CH

    def body(ta, tb, ia, ib, oa, ob, iva, ivb,
             rva0, rva1, rvb0, rvb1, sa0, sa1, sb0, sb1):
        c = lax.axis_index("c")
        s = lax.axis_index("s")
        base = (s * NC + c) * per_w
        rva = (rva0, rva1)
        rvb = (rvb0, rvb1)
        sa = (sa0, sa1)
        sb = (sb0, sb1)
        pltpu.sync_copy(ia.at[pl.ds(base, per_w)], iva)
        pltpu.sync_copy(ib.at[pl.ds(base, per_w)], ivb)

        def issue(j, p):
            pltpu.async_copy(ta.at[iva.at[pl.ds(j * CH, CH)]], rva[p], sa[p])
            pltpu.async_copy(tb.at[ivb.at[pl.ds(j * CH, CH)]], rvb[p], sb[p])

        def drain(j, p):
            pltpu.make_async_copy(ta, rva[p], sa[p]).wait()
            pltpu.make_async_copy(tb, rvb[p], sb[p]).wait()
            off = base + j * CH
            pltpu.sync_copy(rva[p], oa.at[pl.ds(off, CH)])
            pltpu.sync_copy(rvb[p], ob.at[pl.ds(off, CH)])

        issue(0, 0)
        issue(1, 1)

        def outer(i2, carry):
            for p in (0, 1):
                j = i2 * 2 + p
                drain(j, p)
                issue(j + 2, p)
            return carry

        lax.fori_loop(0, nchunk // 2 - 1, outer, 0)
        for p in (0, 1):
            drain(nchunk - 2 + p, p)

    f = pl.kernel(
        body,
        out_type=(
            jax.ShapeDtypeStruct((n, d), _F32),
            jax.ShapeDtypeStruct((n, d), _F32),
        ),
        mesh=_sc_mesh(),
        scratch_types=[
            pltpu.VMEM((per_w,), jnp.int32),
            pltpu.VMEM((per_w,), jnp.int32),
            pltpu.VMEM((CH, d), _F32),
            pltpu.VMEM((CH, d), _F32),
            pltpu.VMEM((CH, d), _F32),
            pltpu.VMEM((CH, d), _F32),
            pltpu.SemaphoreType.DMA,
            pltpu.SemaphoreType.DMA,
            pltpu.SemaphoreType.DMA,
            pltpu.SemaphoreType.DMA,
        ],
    )
    return f(tab_a, tab_b, idx_a, idx_b)


def _segment_sum(vals1, vals2, idx, zeros_init):
    per_h = E_HALF // NW
    nch_h = per_h // CH

    def body(vh1, vh2, ih, zh, oh, iv0, iv1, rv0, rv1, sm0, sm1, acc):
        c = lax.axis_index("c")
        s = lax.axis_index("s")
        r0 = s * ROWS_PER_TILE
        pltpu.sync_copy(zh.at[pl.ds(r0, ROWS_PER_TILE)], acc.at[pl.ds(r0, ROWS_PER_TILE)])
        plsc.subcore_barrier()

        base = (s * NC + c) * per_h
        vh = (vh1, vh2)
        iv = (iv0, iv1)
        rv = (rv0, rv1)
        sm = (sm0, sm1)

        def issue(j, p):
            off = base + j * CH
            pltpu.sync_copy(ih.at[pl.ds(off + p * E_HALF, CH)], iv[p])
            pltpu.async_copy(vh[p].at[pl.ds(off, CH)], rv[p], sm[p])

        def drain(p):
            pltpu.make_async_copy(vh[p].at[pl.ds(0, CH)], rv[p], sm[p]).wait()
            pltpu.sync_copy(rv[p], acc.at[iv[p]], add=True)

        issue(0, 0)
        issue(0, 1)

        def outer(i2, carry):
            for p in (0, 1):
                drain(p)
                issue(i2 + 1, p)
            return carry

        lax.fori_loop(0, nch_h - 1, outer, 0)
        for p in (0, 1):
            drain(p)
        plsc.subcore_barrier()
        pltpu.sync_copy(
            acc.at[pl.ds(r0, ROWS_PER_TILE)],
            oh.at[pl.ds(c * N_PAD + r0, ROWS_PER_TILE)],
        )

    f = pl.kernel(
        body,
        out_type=jax.ShapeDtypeStruct((NC * N_PAD, HID), _F32),
        mesh=_sc_mesh(),
        scratch_types=[
            pltpu.VMEM((CH,), jnp.int32),
            pltpu.VMEM((CH,), jnp.int32),
            pltpu.VMEM((CH, HID), _F32),
            pltpu.VMEM((CH, HID), _F32),
            pltpu.SemaphoreType.DMA,
            pltpu.SemaphoreType.DMA,
            pltpu.VMEM_SHARED((N_PAD, HID), _F32),
        ],
    )
    return f(vals1, vals2, idx, zeros_init)


def _dot(a, b):
    ah = a.astype(jnp.bfloat16)
    al = (a - ah.astype(_F32)).astype(jnp.bfloat16)
    bh = b.astype(jnp.bfloat16)
    bl = (b - bh.astype(_F32)).astype(jnp.bfloat16)
    d = lambda x, y: jnp.dot(x, y, preferred_element_type=_F32)
    return d(ah, bh) + (d(ah, bl) + d(al, bh))


def _mlp_tail(h, w1, w2, b1, b2, g, be):
    h = jnp.maximum(h, 0.0)
    h = jnp.maximum(_dot(h, w1) + b1, 0.0)
    h = _dot(h, w2) + b2
    mu = jnp.mean(h, axis=-1, keepdims=True)
    var = jnp.mean((h - mu) * (h - mu), axis=-1, keepdims=True)
    return (h - mu) * lax.rsqrt(var + EPS) * g + be


def _pack_bias(*rows):
    mat = jnp.stack(list(rows) + [jnp.zeros((HID,), _F32)] * (8 - len(rows)))
    return mat


_BE = 2048
_BN = 1000


def _full(shape):
    return pl.BlockSpec(shape, lambda i: tuple(0 for _ in shape))


def _rows(shape):
    return pl.BlockSpec(shape, lambda i: (i,) + tuple(0 for _ in shape[1:]))


def _edge_enc(ps, pr, w0p, w1, w2, bias):
    def body(ps_ref, pr_ref, w0_ref, w1_ref, w2_ref, bb_ref, o_ref):
        dx = (ps_ref[:, 0:1] - pr_ref[:, 0:1]) * (1.0 / RADIUS)
        dy = (ps_ref[:, 1:2] - pr_ref[:, 1:2]) * (1.0 / RADIUS)
        dist = jnp.sqrt(dx * dx + dy * dy)
        w0 = w0_ref[...]
        bb = bb_ref[...]
        h = dx * w0[0:1] + dy * w0[1:2] + dist * w0[2:3] + bb[0:1]
        o_ref[...] = _mlp_tail(h, w1_ref[...], w2_ref[...], bb[1:2], bb[2:3],
                               bb[3:4], bb[4:5])

    return pl.pallas_call(
        body,
        grid=(E_HALF // _BE,),
        in_specs=[
            _rows((_BE, HID)),
            _rows((_BE, HID)),
            _full((8, HID)),
            _full((HID, HID)),
            _full((HID, HID)),
            _full((8, HID)),
        ],
        out_specs=_rows((_BE, HID)),
        out_shape=jax.ShapeDtypeStruct((E_HALF, HID), _F32),
    )(ps, pr, w0p, w1, w2, bias)


def _node_enc(x, w0p, w1, w2, bias, ws_next, wr_next):
    def body(x_ref, w0_ref, w1_ref, w2_ref, bb_ref, ws_ref, wr_ref,
             on_ref, os_ref, or_ref):
        bb = bb_ref[...]
        h = _dot(x_ref[...], w0_ref[...]) + bb[0:1]
        y = _mlp_tail(h, w1_ref[...], w2_ref[...], bb[1:2], bb[2:3],
                      bb[3:4], bb[4:5])
        on_ref[...] = y
        os_ref[...] = _dot(y, ws_ref[...])
        or_ref[...] = _dot(y, wr_ref[...])

    sds = jax.ShapeDtypeStruct((N_NODES, HID), _F32)
    return pl.pallas_call(
        body,
        grid=(N_NODES // _BN,),
        in_specs=[
            _rows((_BN, 24)),
            _full((24, HID)),
            _full((HID, HID)),
            _full((HID, HID)),
            _full((8, HID)),
            _full((HID, HID)),
            _full((HID, HID)),
        ],
        out_specs=[_rows((_BN, HID))] * 3,
        out_shape=[sds, sds, sds],
    )(x, w0p, w1, w2, bias, ws_next, wr_next)


def _edge_update(edges, gs, gr, we, w1, w2, bias, row0):
    def body(e_ref, gs_ref, gr_ref, we_ref, w1_ref, w2_ref, bb_ref, o_ref):
        bb = bb_ref[...]
        x = e_ref[...]
        h = _dot(x, we_ref[...]) + gs_ref[...] + gr_ref[...] + bb[0:1]
        y = _mlp_tail(h, w1_ref[...], w2_ref[...], bb[1:2], bb[2:3],
                      bb[3:4], bb[4:5])
        out = x + y
        row = lax.broadcasted_iota(jnp.int32, (_BE, 1), 0) + (
            row0 + pl.program_id(0) * _BE)
        o_ref[...] = jnp.where(row < N_EDGES, out, 0.0)

    return pl.pallas_call(
        body,
        grid=(E_HALF // _BE,),
        in_specs=[
            _rows((_BE, HID)),
            _rows((_BE, HID)),
            _rows((_BE, HID)),
            _full((HID, HID)),
            _full((HID, HID)),
            _full((HID, HID)),
            _full((8, HID)),
        ],
        out_specs=_rows((_BE, HID)),
        out_shape=jax.ShapeDtypeStruct((E_HALF, HID), _F32),
    )(edges, gs, gr, we, w1, w2, bias)


def _node_update(nodes, agg, wn, wa, w1, w2, bias, ws_next, wr_next):
    def body(n_ref, a_ref, wn_ref, wa_ref, w1_ref, w2_ref, bb_ref,
             ws_ref, wr_ref, on_ref, os_ref, or_ref):
        bb = bb_ref[...]
        x = n_ref[...]
        a = a_ref[0] + a_ref[1]
        h = _dot(x, wn_ref[...]) + _dot(a, wa_ref[...]) + bb[0:1]
        y = _mlp_tail(h, w1_ref[...], w2_ref[...], bb[1:2], bb[2:3],
                      bb[3:4], bb[4:5])
        nn = x + y
        on_ref[...] = nn
        os_ref[...] = _dot(nn, ws_ref[...])
        or_ref[...] = _dot(nn, wr_ref[...])

    sds = jax.ShapeDtypeStruct((N_NODES, HID), _F32)
    return pl.pallas_call(
        body,
        grid=(N_NODES // _BN,),
        in_specs=[
            _rows((_BN, HID)),
            pl.BlockSpec((NC, _BN, HID), lambda i: (0, i, 0)),
            _full((HID, HID)),
            _full((HID, HID)),
            _full((HID, HID)),
            _full((HID, HID)),
            _full((8, HID)),
            _full((HID, HID)),
            _full((HID, HID)),
        ],
        out_specs=[_rows((_BN, HID))] * 3,
        out_shape=[sds, sds, sds],
    )(nodes, agg, wn, wa, w1, w2, bias, ws_next, wr_next)


def _node_update_dec(nodes, agg, wn, wa, w1, w2, bias, d0, d1, d2p, dbias):
    def body(n_ref, a_ref, wn_ref, wa_ref, w1_ref, w2_ref, bb_ref,
             d0_ref, d1_ref, d2_ref, db_ref, o_ref):
        bb = bb_ref[...]
        db = db_ref[...]
        x = n_ref[...]
        a = a_ref[0] + a_ref[1]
        h = _dot(x, wn_ref[...]) + _dot(a, wa_ref[...]) + bb[0:1]
        y = _mlp_tail(h, w1_ref[...], w2_ref[...], bb[1:2], bb[2:3],
                      bb[3:4], bb[4:5])
        nn = x + y
        h = jnp.maximum(_dot(nn, d0_ref[...]) + db[0:1], 0.0)
        h = jnp.maximum(_dot(h, d1_ref[...]) + db[1:2], 0.0)
        o_ref[...] = _dot(h, d2_ref[...]) + db[2:3]

    return pl.pallas_call(
        body,
        grid=(N_NODES // _BN,),
        in_specs=[
            _rows((_BN, HID)),
            pl.BlockSpec((NC, _BN, HID), lambda i: (0, i, 0)),
            _full((HID, HID)),
            _full((HID, HID)),
            _full((HID, HID)),
            _full((HID, HID)),
            _full((8, HID)),
            _full((HID, HID)),
            _full((HID, HID)),
            _full((HID, HID)),
            _full((8, HID)),
        ],
        out_specs=_rows((_BN, HID)),
        out_shape=jax.ShapeDtypeStruct((N_NODES, HID), _F32),
    )(nodes, agg, wn, wa, w1, w2, bias, d0, d1, d2p, dbias)


def kernel(poss, particle_type, bounds, nonk_mask, tgt_poss, senders,
           receivers, params, num_rollouts):
    poss = poss + (jnp.asarray(num_rollouts) * 0).astype(poss.dtype)
    pos_last = poss[:, -1]

    vels = (poss[:, 1:] - poss[:, :-1]).reshape(N_NODES, (N_HIS - 1) * 2)
    d2w = jnp.concatenate([pos_last - bounds[:, 0], -pos_last + bounds[:, 1]], axis=1)
    d2w = jnp.clip(d2w / RADIUS, -1.0, 1.0)
    onehot = (particle_type[:, None] == jnp.arange(NTYPES)[None, :]).astype(_F32)
    x24 = jnp.concatenate([vels, d2w, onehot, jnp.zeros((N_NODES, 1), _F32)], axis=1)

    ne = params['node_enc']
    w0_eff = jnp.concatenate(
        [ne['W0'][:14], params['emb'] @ ne['W0'][14:30], jnp.zeros((1, HID), _F32)], axis=0)
    ne_bias = _pack_bias(ne['b0'], ne['b1'], ne['b2'], ne['g'], ne['be'])

    pad = jnp.zeros((E_PAD - N_EDGES,), jnp.int32)
    snd = jnp.concatenate([senders.astype(jnp.int32), pad])
    rcv = jnp.concatenate([receivers.astype(jnp.int32), pad])
    snd1, snd2 = snd[:E_HALF], snd[E_HALF:]
    rcv1, rcv2 = rcv[:E_HALF], rcv[E_HALF:]

    pos128 = jnp.pad(pos_last, ((0, 0), (0, HID - 2)))
    ps1, pr1 = _gather_pair(pos128, pos128, snd1, rcv1)
    ps2, pr2 = _gather_pair(pos128, pos128, snd2, rcv2)

    ee = params['edge_enc']
    ee_w0 = jnp.concatenate([ee['W0'], jnp.zeros((5, HID), _F32)], axis=0)
    ee_bias = _pack_bias(ee['b0'], ee['b1'], ee['b2'], ee['g'], ee['be'])
    edges1 = _edge_enc(ps1, pr1, ee_w0, ee['W1'], ee['W2'], ee_bias)
    edges2 = _edge_enc(ps2, pr2, ee_w0, ee['W1'], ee['W2'], ee_bias)

    g0 = params['gnn'][0]['edge']['W0']
    nodes, S, R = _node_enc(x24, w0_eff, ne['W1'], ne['W2'], ne_bias,
                            g0[HID:2 * HID], g0[2 * HID:])

    zrows = jnp.zeros((N_PAD, HID), _F32)
    pred128 = None
    for l in range(LAYERS):
        lw = params['gnn'][l]
        ew = lw['edge']
        e_bias = _pack_bias(ew['b0'], ew['b1'], ew['b2'], ew['g'], ew['be'])
        gs1, gr1 = _gather_pair(S, R, snd1, rcv1)
        gs2, gr2 = _gather_pair(S, R, snd2, rcv2)
        edges1 = _edge_update(edges1, gs1, gr1, ew['W0'][:HID], ew['W1'],
                              ew['W2'], e_bias, 0)
        edges2 = _edge_update(edges2, gs2, gr2, ew['W0'][:HID], ew['W1'],
                              ew['W2'], e_bias, E_HALF)
        aggf = _segment_sum(edges1, edges2, rcv, zrows)
        agg = aggf.reshape(NC, N_PAD, HID)
        nw = lw['node']
        n_bias = _pack_bias(nw['b0'], nw['b1'], nw['b2'], nw['g'], nw['be'])
        if l < LAYERS - 1:
            gnext = params['gnn'][l + 1]['edge']['W0']
            nodes, S, R = _node_update(nodes, agg, nw['W0'][:HID], nw['W0'][HID:],
                                       nw['W1'], nw['W2'], n_bias,
                                       gnext[HID:2 * HID], gnext[2 * HID:])
        else:
            dec = params['dec']
            d2p = jnp.pad(dec['W2'], ((0, 0), (0, HID - 2)))
            db2p = jnp.pad(dec['b2'], (0, HID - 2))
            d_bias = _pack_bias(dec['b0'], dec['b1'], db2p)
            pred128 = _node_update_dec(nodes, agg, nw['W0'][:HID], nw['W0'][HID:],
                                       nw['W1'], nw['W2'], n_bias,
                                       dec['W0'], dec['W1'], d2p, d_bias)

    pred_acc = pred128[:, :2]

    pred_vel = poss[:, -1] - poss[:, -2]
    pred_pos = poss[:, -1] + pred_vel + pred_acc
    pred_pos = jnp.where(nonk_mask[:, None].astype(bool), pred_pos, tgt_poss[:, 0])
    pred_accns = pred_acc[:, None, :]
    pred_poss = pred_pos[:, None, :]
    return pred_accns, pred_poss


# speedup vs baseline: 2.4666x; 1.0031x over previous
import functools

import jax
import jax.numpy as jnp
from jax import lax
from jax.experimental import pallas as pl
from jax.experimental.pallas import tpu as pltpu
from jax.experimental.pallas import tpu_sc as plsc

N_NODES = 10000
N_EDGES = 160000
N_HIS = 6
HID = 128
EMB = 16
NTYPES = 9
LAYERS = 5
RADIUS = 0.1
EPS = 1e-5

NC = 2
NS = 16
NW = NC * NS

CH = 128
E_PAD = 163840
E_HALF = E_PAD // 2

N_PAD = 10240
ROWS_PER_TILE = N_PAD // NS

_F32 = jnp.float32


def _sc_mesh():
    return plsc.VectorSubcoreMesh(
        core_axis_name="c", subcore_axis_name="s", num_cores=NC, num_subcores=NS
    )


def _gather_pair(tab_a, tab_b, idx_a, idx_b):
    d = tab_a.shape[-1]
    n = idx_a.shape[0]
    per_w = n // NW
    nchunk = per_w // CH

    ns = 3
    assert nchunk % ns == 2, nchunk

    def body(ta, tb, ia, ib, oa, ob, iva, ivb,
             rva0, rva1, rva2, rvb0, rvb1, rvb2,
             sa0, sa1, sa2, sb0, sb1, sb2):
        c = lax.axis_index("c")
        s = lax.axis_index("s")
        base = (s * NC + c) * per_w
        rva = (rva0, rva1, rva2)
        rvb = (rvb0, rvb1, rvb2)
        sa = (sa0, sa1, sa2)
        sb = (sb0, sb1, sb2)
        pltpu.sync_copy(ia.at[pl.ds(base, per_w)], iva)
        pltpu.sync_copy(ib.at[pl.ds(base, per_w)], ivb)

        def issue(j, p):
            pltpu.async_copy(ta.at[iva.at[pl.ds(j * CH, CH)]], rva[p], sa[p])
            pltpu.async_copy(tb.at[ivb.at[pl.ds(j * CH, CH)]], rvb[p], sb[p])

        def drain(j, p):
            pltpu.make_async_copy(ta, rva[p], sa[p]).wait()
            pltpu.make_async_copy(tb, rvb[p], sb[p]).wait()
            off = base + j * CH
            pltpu.sync_copy(rva[p], oa.at[pl.ds(off, CH)])
            pltpu.sync_copy(rvb[p], ob.at[pl.ds(off, CH)])

        for p in range(ns):
            issue(p, p)

        def outer(i, carry):
            for p in range(ns):
                j = i * ns + p
                drain(j, p)
                issue(j + ns, p)
            return carry

        lax.fori_loop(0, nchunk // ns - 1, outer, 0)
        tail0 = ns * (nchunk // ns - 1)
        for j in range(tail0, nchunk):
            p = j % ns
            drain(j, p)
            if j + ns < nchunk:
                issue(j + ns, p)

    f = pl.kernel(
        body,
        out_type=(
            jax.ShapeDtypeStruct((n, d), _F32),
            jax.ShapeDtypeStruct((n, d), _F32),
        ),
        mesh=_sc_mesh(),
        scratch_types=[
            pltpu.VMEM((per_w,), jnp.int32),
            pltpu.VMEM((per_w,), jnp.int32),
        ] + [pltpu.VMEM((CH, d), _F32)] * 6
          + [pltpu.SemaphoreType.DMA] * 6,
    )
    return f(tab_a, tab_b, idx_a, idx_b)


def _segment_sum(vals1, vals2, idx, zeros_init):
    per_h = E_HALF // NW
    nch_h = per_h // CH

    def body(vh1, vh2, ih, zh, oh, iv0, iv1, rv0, rv1, sm0, sm1, acc):
        c = lax.axis_index("c")
        s = lax.axis_index("s")
        r0 = s * ROWS_PER_TILE
        pltpu.sync_copy(zh.at[pl.ds(r0, ROWS_PER_TILE)], acc.at[pl.ds(r0, ROWS_PER_TILE)])
        plsc.subcore_barrier()

        base = (s * NC + c) * per_h
        vh = (vh1, vh2)
        iv = (iv0, iv1)
        rv = (rv0, rv1)
        sm = (sm0, sm1)

        def issue(j, p):
            off = base + j * CH
            pltpu.sync_copy(ih.at[pl.ds(off + p * E_HALF, CH)], iv[p])
            pltpu.async_copy(vh[p].at[pl.ds(off, CH)], rv[p], sm[p])

        def drain(p):
            pltpu.make_async_copy(vh[p].at[pl.ds(0, CH)], rv[p], sm[p]).wait()
            pltpu.sync_copy(rv[p], acc.at[iv[p]], add=True)

        issue(0, 0)
        issue(0, 1)

        def outer(i2, carry):
            for p in (0, 1):
                drain(p)
                issue(i2 + 1, p)
            return carry

        lax.fori_loop(0, nch_h - 1, outer, 0)
        for p in (0, 1):
            drain(p)
        plsc.subcore_barrier()
        pltpu.sync_copy(
            acc.at[pl.ds(r0, ROWS_PER_TILE)],
            oh.at[pl.ds(c * N_PAD + r0, ROWS_PER_TILE)],
        )

    f = pl.kernel(
        body,
        out_type=jax.ShapeDtypeStruct((NC * N_PAD, HID), _F32),
        mesh=_sc_mesh(),
        scratch_types=[
            pltpu.VMEM((CH,), jnp.int32),
            pltpu.VMEM((CH,), jnp.int32),
            pltpu.VMEM((CH, HID), _F32),
            pltpu.VMEM((CH, HID), _F32),
            pltpu.SemaphoreType.DMA,
            pltpu.SemaphoreType.DMA,
            pltpu.VMEM_SHARED((N_PAD, HID), _F32),
        ],
    )
    return f(vals1, vals2, idx, zeros_init)


def _dot(a, b):
    ah = a.astype(jnp.bfloat16)
    al = (a - ah.astype(_F32)).astype(jnp.bfloat16)
    bh = b.astype(jnp.bfloat16)
    bl = (b - bh.astype(_F32)).astype(jnp.bfloat16)
    d = lambda x, y: jnp.dot(x, y, preferred_element_type=_F32)
    return d(ah, bh) + (d(ah, bl) + d(al, bh))


def _mlp_tail(h, w1, w2, b1, b2, g, be):
    h = jnp.maximum(h, 0.0)
    h = jnp.maximum(_dot(h, w1) + b1, 0.0)
    h = _dot(h, w2) + b2
    mu = jnp.mean(h, axis=-1, keepdims=True)
    var = jnp.mean((h - mu) * (h - mu), axis=-1, keepdims=True)
    return (h - mu) * lax.rsqrt(var + EPS) * g + be


def _pack_bias(*rows):
    mat = jnp.stack(list(rows) + [jnp.zeros((HID,), _F32)] * (8 - len(rows)))
    return mat


_BE = 2048
_BN = 1000


def _full(shape):
    return pl.BlockSpec(shape, lambda i: tuple(0 for _ in shape))


def _rows(shape):
    return pl.BlockSpec(shape, lambda i: (i,) + tuple(0 for _ in shape[1:]))


def _edge_enc(ps, pr, w0p, w1, w2, bias):
    def body(ps_ref, pr_ref, w0_ref, w1_ref, w2_ref, bb_ref, o_ref):
        dx = (ps_ref[:, 0:1] - pr_ref[:, 0:1]) * (1.0 / RADIUS)
        dy = (ps_ref[:, 1:2] - pr_ref[:, 1:2]) * (1.0 / RADIUS)
        dist = jnp.sqrt(dx * dx + dy * dy)
        w0 = w0_ref[...]
        bb = bb_ref[...]
        h = dx * w0[0:1] + dy * w0[1:2] + dist * w0[2:3] + bb[0:1]
        o_ref[...] = _mlp_tail(h, w1_ref[...], w2_ref[...], bb[1:2], bb[2:3],
                               bb[3:4], bb[4:5])

    return pl.pallas_call(
        body,
        grid=(E_HALF // _BE,),
        in_specs=[
            _rows((_BE, HID)),
            _rows((_BE, HID)),
            _full((8, HID)),
            _full((HID, HID)),
            _full((HID, HID)),
            _full((8, HID)),
        ],
        out_specs=_rows((_BE, HID)),
        out_shape=jax.ShapeDtypeStruct((E_HALF, HID), _F32),
    )(ps, pr, w0p, w1, w2, bias)


def _node_enc(x, w0p, w1, w2, bias, ws_next, wr_next):
    def body(x_ref, w0_ref, w1_ref, w2_ref, bb_ref, ws_ref, wr_ref,
             on_ref, os_ref, or_ref):
        bb = bb_ref[...]
        h = _dot(x_ref[...], w0_ref[...]) + bb[0:1]
        y = _mlp_tail(h, w1_ref[...], w2_ref[...], bb[1:2], bb[2:3],
                      bb[3:4], bb[4:5])
        on_ref[...] = y
        os_ref[...] = _dot(y, ws_ref[...])
        or_ref[...] = _dot(y, wr_ref[...])

    sds = jax.ShapeDtypeStruct((N_NODES, HID), _F32)
    return pl.pallas_call(
        body,
        grid=(N_NODES // _BN,),
        in_specs=[
            _rows((_BN, 24)),
            _full((24, HID)),
            _full((HID, HID)),
            _full((HID, HID)),
            _full((8, HID)),
            _full((HID, HID)),
            _full((HID, HID)),
        ],
        out_specs=[_rows((_BN, HID))] * 3,
        out_shape=[sds, sds, sds],
    )(x, w0p, w1, w2, bias, ws_next, wr_next)


def _edge_update(edges, gs, gr, we, w1, w2, bias, row0):
    def body(e_ref, gs_ref, gr_ref, we_ref, w1_ref, w2_ref, bb_ref, o_ref):
        bb = bb_ref[...]
        x = e_ref[...]
        h = _dot(x, we_ref[...]) + gs_ref[...] + gr_ref[...] + bb[0:1]
        y = _mlp_tail(h, w1_ref[...], w2_ref[...], bb[1:2], bb[2:3],
                      bb[3:4], bb[4:5])
        out = x + y
        row = lax.broadcasted_iota(jnp.int32, (_BE, 1), 0) + (
            row0 + pl.program_id(0) * _BE)
        o_ref[...] = jnp.where(row < N_EDGES, out, 0.0)

    return pl.pallas_call(
        body,
        grid=(E_HALF // _BE,),
        in_specs=[
            _rows((_BE, HID)),
            _rows((_BE, HID)),
            _rows((_BE, HID)),
            _full((HID, HID)),
            _full((HID, HID)),
            _full((HID, HID)),
            _full((8, HID)),
        ],
        out_specs=_rows((_BE, HID)),
        out_shape=jax.ShapeDtypeStruct((E_HALF, HID), _F32),
    )(edges, gs, gr, we, w1, w2, bias)


def _node_update(nodes, agg, wn, wa, w1, w2, bias, ws_next, wr_next):
    def body(n_ref, a_ref, wn_ref, wa_ref, w1_ref, w2_ref, bb_ref,
             ws_ref, wr_ref, on_ref, os_ref, or_ref):
        bb = bb_ref[...]
        x = n_ref[...]
        a = a_ref[0] + a_ref[1]
        h = _dot(x, wn_ref[...]) + _dot(a, wa_ref[...]) + bb[0:1]
        y = _mlp_tail(h, w1_ref[...], w2_ref[...], bb[1:2], bb[2:3],
                      bb[3:4], bb[4:5])
        nn = x + y
        on_ref[...] = nn
        os_ref[...] = _dot(nn, ws_ref[...])
        or_ref[...] = _dot(nn, wr_ref[...])

    sds = jax.ShapeDtypeStruct((N_NODES, HID), _F32)
    return pl.pallas_call(
        body,
        grid=(N_NODES // _BN,),
        in_specs=[
            _rows((_BN, HID)),
            pl.BlockSpec((NC, _BN, HID), lambda i: (0, i, 0)),
            _full((HID, HID)),
            _full((HID, HID)),
            _full((HID, HID)),
            _full((HID, HID)),
            _full((8, HID)),
            _full((HID, HID)),
            _full((HID, HID)),
        ],
        out_specs=[_rows((_BN, HID))] * 3,
        out_shape=[sds, sds, sds],
    )(nodes, agg, wn, wa, w1, w2, bias, ws_next, wr_next)


def _node_update_dec(nodes, agg, wn, wa, w1, w2, bias, d0, d1, d2p, dbias):
    def body(n_ref, a_ref, wn_ref, wa_ref, w1_ref, w2_ref, bb_ref,
             d0_ref, d1_ref, d2_ref, db_ref, o_ref):
        bb = bb_ref[...]
        db = db_ref[...]
        x = n_ref[...]
        a = a_ref[0] + a_ref[1]
        h = _dot(x, wn_ref[...]) + _dot(a, wa_ref[...]) + bb[0:1]
        y = _mlp_tail(h, w1_ref[...], w2_ref[...], bb[1:2], bb[2:3],
                      bb[3:4], bb[4:5])
        nn = x + y
        h = jnp.maximum(_dot(nn, d0_ref[...]) + db[0:1], 0.0)
        h = jnp.maximum(_dot(h, d1_ref[...]) + db[1:2], 0.0)
        o_ref[...] = _dot(h, d2_ref[...]) + db[2:3]

    return pl.pallas_call(
        body,
        grid=(N_NODES // _BN,),
        in_specs=[
            _rows((_BN, HID)),
            pl.BlockSpec((NC, _BN, HID), lambda i: (0, i, 0)),
            _full((HID, HID)),
            _full((HID, HID)),
            _full((HID, HID)),
            _full((HID, HID)),
            _full((8, HID)),
            _full((HID, HID)),
            _full((HID, HID)),
            _full((HID, HID)),
            _full((8, HID)),
        ],
        out_specs=_rows((_BN, HID)),
        out_shape=jax.ShapeDtypeStruct((N_NODES, HID), _F32),
    )(nodes, agg, wn, wa, w1, w2, bias, d0, d1, d2p, dbias)


def kernel(poss, particle_type, bounds, nonk_mask, tgt_poss, senders,
           receivers, params, num_rollouts):
    poss = poss + (jnp.asarray(num_rollouts) * 0).astype(poss.dtype)
    pos_last = poss[:, -1]

    vels = (poss[:, 1:] - poss[:, :-1]).reshape(N_NODES, (N_HIS - 1) * 2)
    d2w = jnp.concatenate([pos_last - bounds[:, 0], -pos_last + bounds[:, 1]], axis=1)
    d2w = jnp.clip(d2w / RADIUS, -1.0, 1.0)
    onehot = (particle_type[:, None] == jnp.arange(NTYPES)[None, :]).astype(_F32)
    x24 = jnp.concatenate([vels, d2w, onehot, jnp.zeros((N_NODES, 1), _F32)], axis=1)

    ne = params['node_enc']
    w0_eff = jnp.concatenate(
        [ne['W0'][:14], params['emb'] @ ne['W0'][14:30], jnp.zeros((1, HID), _F32)], axis=0)
    ne_bias = _pack_bias(ne['b0'], ne['b1'], ne['b2'], ne['g'], ne['be'])

    pad = jnp.zeros((E_PAD - N_EDGES,), jnp.int32)
    snd = jnp.concatenate([senders.astype(jnp.int32), pad])
    rcv = jnp.concatenate([receivers.astype(jnp.int32), pad])
    snd1, snd2 = snd[:E_HALF], snd[E_HALF:]
    rcv1, rcv2 = rcv[:E_HALF], rcv[E_HALF:]

    pos128 = jnp.pad(pos_last, ((0, 0), (0, HID - 2)))
    ps1, pr1 = _gather_pair(pos128, pos128, snd1, rcv1)
    ps2, pr2 = _gather_pair(pos128, pos128, snd2, rcv2)

    ee = params['edge_enc']
    ee_w0 = jnp.concatenate([ee['W0'], jnp.zeros((5, HID), _F32)], axis=0)
    ee_bias = _pack_bias(ee['b0'], ee['b1'], ee['b2'], ee['g'], ee['be'])
    edges1 = _edge_enc(ps1, pr1, ee_w0, ee['W1'], ee['W2'], ee_bias)
    edges2 = _edge_enc(ps2, pr2, ee_w0, ee['W1'], ee['W2'], ee_bias)

    g0 = params['gnn'][0]['edge']['W0']
    nodes, S, R = _node_enc(x24, w0_eff, ne['W1'], ne['W2'], ne_bias,
                            g0[HID:2 * HID], g0[2 * HID:])

    zrows = jnp.zeros((N_PAD, HID), _F32)
    pred128 = None
    for l in range(LAYERS):
        lw = params['gnn'][l]
        ew = lw['edge']
        e_bias = _pack_bias(ew['b0'], ew['b1'], ew['b2'], ew['g'], ew['be'])
        gs1, gr1 = _gather_pair(S, R, snd1, rcv1)
        gs2, gr2 = _gather_pair(S, R, snd2, rcv2)
        edges1 = _edge_update(edges1, gs1, gr1, ew['W0'][:HID], ew['W1'],
                              ew['W2'], e_bias, 0)
        edges2 = _edge_update(edges2, gs2, gr2, ew['W0'][:HID], ew['W1'],
                              ew['W2'], e_bias, E_HALF)
        aggf = _segment_sum(edges1, edges2, rcv, zrows)
        agg = aggf.reshape(NC, N_PAD, HID)
        nw = lw['node']
        n_bias = _pack_bias(nw['b0'], nw['b1'], nw['b2'], nw['g'], nw['be'])
        if l < LAYERS - 1:
            gnext = params['gnn'][l + 1]['edge']['W0']
            nodes, S, R = _node_update(nodes, agg, nw['W0'][:HID], nw['W0'][HID:],
                                       nw['W1'], nw['W2'], n_bias,
                                       gnext[HID:2 * HID], gnext[2 * HID:])
        else:
            dec = params['dec']
            d2p = jnp.pad(dec['W2'], ((0, 0), (0, HID - 2)))
            db2p = jnp.pad(dec['b2'], (0, HID - 2))
            d_bias = _pack_bias(dec['b0'], dec['b1'], db2p)
            pred128 = _node_update_dec(nodes, agg, nw['W0'][:HID], nw['W0'][HID:],
                                       nw['W1'], nw['W2'], n_bias,
                                       dec['W0'], dec['W1'], d2p, d_bias)

    pred_acc = pred128[:, :2]

    pred_vel = poss[:, -1] - poss[:, -2]
    pred_pos = poss[:, -1] + pred_vel + pred_acc
    pred_pos = jnp.where(nonk_mask[:, None].astype(bool), pred_pos, tgt_poss[:, 0])
    pred_accns = pred_acc[:, None, :]
    pred_poss = pred_pos[:, None, :]
    return pred_accns, pred_poss
